# Initial kernel scaffold; baseline (speedup 1.0000x reference)
import jax
import jax.numpy as jnp
from jax.experimental import pallas as pl


def kernel(feat_word, feat_topic, effect, ww_src, ww_dst, ww_w, wt_src, wt_dst, wt_w, wd_src, wd_dst, wd_w, td_src, td_dst, td_w, tt_src, tt_dst, tt_w, rand_td, rand_tt, W_ww, b_ww, W_wt, b_wt, W_wd, b_wd, W_td, b_td, W_tt, b_tt, W_td_cau, W_td_noi, W_tt_cau, W_tt_noi, W_td_cau_trans, W_td_noi_trans, W_tt_cau_trans, W_tt_noi_trans):
    raise NotImplementedError("write your pallas kernel here")


# speedup vs baseline: 2.1608x; 2.1608x over previous
import functools
import jax
import jax.numpy as jnp
from jax import lax
from jax.experimental import pallas as pl
from jax.experimental.pallas import tpu as pltpu
from jax.experimental.pallas import tpu_sc as plsc

N_WORD, N_TOPIC, N_DOC = 50000, 5000, 20000
IN_SIZE, OUT_SIZE = 128, 128
B = 512
NTILES = 16
NCORES = 2
D = 128


def _rup(x, m):
    return (x + m - 1) // m * m


NP_WORD = _rup(N_WORD, 1024)
NP_TOPIC = _rup(N_TOPIC, 1024)
NP_DOC = _rup(N_DOC, 1024)

E_WW, E_WT, E_WD, E_TD, E_TT = 300000, 100000, 100000, 50000, 50000


def _proj1_body(x_ref, w_ref, b_ref, o_ref):
    o_ref[...] = jnp.dot(x_ref[...], w_ref[...].T,
                         preferred_element_type=jnp.float32) + b_ref[...]


def _proj_word(x, w, b):
    blk = 400
    return pl.pallas_call(
        _proj1_body,
        grid=(N_WORD // blk,),
        in_specs=[
            pl.BlockSpec((blk, IN_SIZE), lambda i: (i, 0)),
            pl.BlockSpec((OUT_SIZE, IN_SIZE), lambda i: (0, 0)),
            pl.BlockSpec((1, OUT_SIZE), lambda i: (0, 0)),
        ],
        out_specs=pl.BlockSpec((blk, OUT_SIZE), lambda i: (i, 0)),
        out_shape=jax.ShapeDtypeStruct((N_WORD, OUT_SIZE), jnp.float32),
    )(x, w, b.reshape(1, OUT_SIZE))


def _topic_body(x_ref, eff_ref, rtd_ref, rtt_ref,
                wtd_ref, btd_ref, wtdc_ref, wtdn_ref,
                wtt_ref, btt_ref, wttc_ref, wttn_ref,
                ttdc_ref, ttdn_ref, tttc_ref, tttn_ref,
                otd_ref, ott_ref):
    x = x_ref[...]
    eff = eff_ref[...]
    zero = (eff == 0.0).astype(jnp.float32)
    mtd = (rtd_ref[...] < 0.1).astype(jnp.float32) * zero
    mtt = (rtt_ref[...] < 0.1).astype(jnp.float32) * zero

    def trans(m, t_ref):
        t = t_ref[...]
        return (m[:, 0:1] * t[0, 0] + m[:, 1:2] * t[0, 1]
                + m[:, 2:3] * t[0, 2])

    def mm(a, w_ref):
        return jnp.dot(a, w_ref[...].T, preferred_element_type=jnp.float32)

    ew_td = trans(eff, ttdc_ref)
    nw_td = trans(mtd, ttdn_ref)
    ew_tt = trans(eff, tttc_ref)
    nw_tt = trans(mtt, tttn_ref)
    otd_ref[...] = (mm(x, wtd_ref) + btd_ref[...]
                    + mm(x * ew_td, wtdc_ref) - mm(x * nw_td, wtdn_ref))
    ott_ref[...] = (mm(x, wtt_ref) + btt_ref[...]
                    + mm(x * ew_tt, wttc_ref) - mm(x * nw_tt, wttn_ref))


def _topic_proj(x, eff, rtd, rtt, wtd, btd, wtdc, wtdn,
                wtt, btt, wttc, wttn, ttdc, ttdn, tttc, tttn):
    blk = 1000
    mat = lambda: pl.BlockSpec((OUT_SIZE, OUT_SIZE), lambda i: (0, 0))
    tri = lambda: pl.BlockSpec((1, 3), lambda i: (0, 0))
    bia = lambda: pl.BlockSpec((1, OUT_SIZE), lambda i: (0, 0))
    return pl.pallas_call(
        _topic_body,
        grid=(N_TOPIC // blk,),
        in_specs=[
            pl.BlockSpec((blk, OUT_SIZE), lambda i: (i, 0)),
            pl.BlockSpec((blk, 3), lambda i: (i, 0)),
            pl.BlockSpec((blk, 3), lambda i: (i, 0)),
            pl.BlockSpec((blk, 3), lambda i: (i, 0)),
            mat(), bia(), mat(), mat(),
            mat(), bia(), mat(), mat(),
            tri(), tri(), tri(), tri(),
        ],
        out_specs=[pl.BlockSpec((blk, OUT_SIZE), lambda i: (i, 0))] * 2,
        out_shape=[jax.ShapeDtypeStruct((N_TOPIC, OUT_SIZE), jnp.float32)] * 2,
    )(x, eff, rtd, rtt, wtd, btd.reshape(1, -1), wtdc, wtdn,
      wtt, btt.reshape(1, -1), wttc, wttn, ttdc, ttdn, tttc, tttn)


_ONES32 = None


def _cnt_col(c_block):
    ones = jnp.ones((32, 1), jnp.float32)
    return lax.dot_general(c_block, ones, (((0,), (0,)), ((), ())),
                           preferred_element_type=jnp.float32)


def _word2_body(s_ref, c_ref, wwt_ref, bwt_ref, wwd_ref, bwd_ref,
                h_ref, owt_ref, owd_ref):
    c = jnp.maximum(_cnt_col(c_ref[...]), 1.0)
    h = s_ref[...] / c
    h_ref[...] = h
    owt_ref[...] = jnp.dot(h, wwt_ref[...].T,
                           preferred_element_type=jnp.float32) + bwt_ref[...]
    owd_ref[...] = jnp.dot(h, wwd_ref[...].T,
                           preferred_element_type=jnp.float32) + bwd_ref[...]


def _word_pass2(s, cnt, wwt, bwt, wwd, bwd):
    blk = 512
    mat = lambda: pl.BlockSpec((OUT_SIZE, OUT_SIZE), lambda i: (0, 0))
    bia = lambda: pl.BlockSpec((1, OUT_SIZE), lambda i: (0, 0))
    return pl.pallas_call(
        _word2_body,
        grid=(NP_WORD // blk,),
        in_specs=[
            pl.BlockSpec((blk, OUT_SIZE), lambda i: (i, 0)),
            pl.BlockSpec((32, blk), lambda i: (0, i)),
            mat(), bia(), mat(), bia(),
        ],
        out_specs=[pl.BlockSpec((blk, OUT_SIZE), lambda i: (i, 0))] * 3,
        out_shape=[jax.ShapeDtypeStruct((NP_WORD, OUT_SIZE), jnp.float32)] * 3,
    )(s, cnt, wwt, bwt.reshape(1, -1), wwd, bwd.reshape(1, -1))


def _combine_body(sa_ref, ca_ref, sb_ref, cb_ref, o_ref):
    ca = jnp.maximum(_cnt_col(ca_ref[...]), 1.0)
    cb = jnp.maximum(_cnt_col(cb_ref[...]), 1.0)
    o_ref[...] = sa_ref[...] / ca + sb_ref[...] / cb


def _combine(n_pad, sa, ca, sb, cb):
    blk = 512
    return pl.pallas_call(
        _combine_body,
        grid=(n_pad // blk,),
        in_specs=[
            pl.BlockSpec((blk, OUT_SIZE), lambda i: (i, 0)),
            pl.BlockSpec((32, blk), lambda i: (0, i)),
            pl.BlockSpec((blk, OUT_SIZE), lambda i: (i, 0)),
            pl.BlockSpec((32, blk), lambda i: (0, i)),
        ],
        out_specs=pl.BlockSpec((blk, OUT_SIZE), lambda i: (i, 0)),
        out_shape=jax.ShapeDtypeStruct((n_pad, OUT_SIZE), jnp.float32),
    )(sa, ca, sb, cb)


def _mesh():
    return plsc.VectorSubcoreMesh(core_axis_name="c", subcore_axis_name="s")


def _edge_passes(tables, edges, out_shapes, passes, acc_rows, bb):
    nt = len(tables)
    ne = len(edges)
    no = len(out_shapes)

    @functools.partial(
        pl.kernel,
        out_type=[jax.ShapeDtypeStruct((n, D), jnp.float32)
                  for n in out_shapes],
        mesh=_mesh(),
        compiler_params=pltpu.CompilerParams(needs_layout_passes=False),
        scratch_types=[
            pltpu.VMEM_SHARED((acc_rows, D), jnp.float32),
            pltpu.VMEM((bb,), jnp.int32),
            pltpu.VMEM((bb,), jnp.int32),
            pltpu.VMEM((bb,), jnp.float32),
            pltpu.VMEM((bb, D), jnp.float32),
            pltpu.SemaphoreType.DMA,
        ],
    )
    def seg_kernel(*refs):
        tabs = refs[:nt]
        srcs = refs[nt:nt + ne]
        dsts = refs[nt + ne:nt + 2 * ne]
        ws = refs[nt + 2 * ne:nt + 3 * ne]
        outs = refs[nt + 3 * ne:nt + 3 * ne + no]
        acc, src_v, dst_v, w_v, rows_v, sem = refs[nt + 3 * ne + no:]
        cid = lax.axis_index("c")
        sid = lax.axis_index("s")

        for (core, ti, ei, oi, q0, rng, n_e) in passes:
            @pl.when(cid == core)
            def _(ti=ti, ei=ei, oi=oi, q0=q0, rng=rng, n_e=n_e):
                tbl = tabs[ti]
                srch, dsth, wh = srcs[ei], dsts[ei], ws[ei]
                out = outs[oi]
                ep = srch.shape[0]
                nb = ep // (NTILES * bb)
                zrows = rng + 16
                zpt = zrows // NTILES
                rpt = rng // NTILES

                def zr(i, _):
                    for c in range(D // 16):
                        rows_v[i, pl.ds(c * 16, 16)] = (
                            jnp.zeros((16,), jnp.float32))
                    return 0
                lax.fori_loop(0, bb, zr, 0)
                off = 0
                while off < zpt:
                    n = min(bb, zpt - off)
                    pltpu.sync_copy(rows_v.at[pl.ds(0, n)],
                                    acc.at[pl.ds(sid * zpt + off, n)])
                    off += n
                plsc.subcore_barrier()

                def batch(b, _):
                    base = (b * NTILES + sid) * bb

                    @pl.when(base < n_e)
                    def _():
                        pltpu.sync_copy(srch.at[pl.ds(base, bb)], src_v)
                        pltpu.sync_copy(dsth.at[pl.ds(base, bb)], dst_v)
                        pltpu.sync_copy(wh.at[pl.ds(base, bb)], w_v)

                        def remap(g, _):
                            dvec = dst_v[pl.ds(g * 16, 16)] - q0
                            ok = (dvec >= 0) & (dvec < rng)
                            dst_v[pl.ds(g * 16, 16)] = jnp.where(
                                ok, dvec, rng)
                            return 0
                        lax.fori_loop(0, bb // 16, remap, 0)

                        pltpu.async_copy(tbl.at[src_v], rows_v, sem).wait()

                        def scale(g, _):
                            wvec = w_v[pl.ds(g * 16, 16)]
                            for j in range(16):
                                wspl = jnp.broadcast_to(wvec[j], (16,))
                                e = g * 16 + j
                                for c in range(D // 16):
                                    rows_v[e, pl.ds(c * 16, 16)] = (
                                        rows_v[e, pl.ds(c * 16, 16)] * wspl)
                            return 0
                        lax.fori_loop(0, bb // 16, scale, 0)

                        pltpu.sync_copy(rows_v, acc.at[dst_v], add=True)
                    return 0
                lax.fori_loop(0, nb, batch, 0)
                plsc.subcore_barrier()
                pltpu.sync_copy(acc.at[pl.ds(sid * rpt, rpt)],
                                out.at[pl.ds(q0 + sid * rpt, rpt)])
                plsc.subcore_barrier()

    flat = list(tables)
    for i in range(3):
        flat += [e[i] for e in edges]
    res = seg_kernel(*flat)
    return res if isinstance(res, (tuple, list)) else (res,)


def _counts_kernel(dsts_padded, specs):
    ne = len(specs)
    npmax = max(n for (_, n) in specs)

    @functools.partial(
        pl.kernel,
        out_type=[jax.ShapeDtypeStruct((32, n), jnp.float32)
                  for (_, n) in specs],
        mesh=_mesh(),
        compiler_params=pltpu.CompilerParams(needs_layout_passes=False),
        scratch_types=[
            pltpu.VMEM((npmax,), jnp.float32),
            pltpu.VMEM((B,), jnp.int32),
        ],
    )
    def cnt_kernel(*refs):
        dsts = refs[:ne]
        outs = refs[ne:2 * ne]
        plane, dst_v = refs[2 * ne:]
        cid = lax.axis_index("c")
        sid = lax.axis_index("s")
        wid = sid * NCORES + cid
        lane = lax.iota(jnp.int32, 16)
        ones = jnp.ones((16,), jnp.float32)

        for k, (n_e, n_pad) in enumerate(specs):
            dsth = dsts[k]
            out = outs[k]
            ep = dsth.shape[0]
            nb = ep // (32 * B)

            def zp(i, _):
                plane[pl.ds(i * 16, 16)] = jnp.zeros((16,), jnp.float32)
                return 0
            lax.fori_loop(0, n_pad // 16, zp, 0)

            def batch(b, _):
                base = (b * 32 + wid) * B

                @pl.when(base < n_e)
                def _():
                    pltpu.sync_copy(dsth.at[pl.ds(base, B)], dst_v)

                    def grp(g, _):
                        dvec = dst_v[pl.ds(g * 16, 16)]
                        valid = (base + g * 16 + lane) < n_e
                        for l in range(16):
                            m = valid & (lane == l)
                            plsc.addupdate_scatter(
                                plane, [dvec], ones, mask=m)
                        return 0
                    lax.fori_loop(0, B // 16, grp, 0)
                return 0
            lax.fori_loop(0, nb, batch, 0)
            pltpu.sync_copy(plane.at[pl.ds(0, n_pad)], out.at[wid])

    res = cnt_kernel(*dsts_padded)
    return res if isinstance(res, (tuple, list)) else (res,)


def _pad_edges(src, dst, w):
    e = src.shape[0]
    ep = _rup(e, 32 * B)
    pad = ep - e
    src_p = jnp.pad(src.astype(jnp.int32), (0, pad))
    dst_p = jnp.pad(dst.astype(jnp.int32), (0, pad))
    w_p = jnp.pad(w, (0, pad))
    return src_p, dst_p, w_p


def kernel(feat_word, feat_topic, effect,
           ww_src, ww_dst, ww_w, wt_src, wt_dst, wt_w,
           wd_src, wd_dst, wd_w, td_src, td_dst, td_w,
           tt_src, tt_dst, tt_w, rand_td, rand_tt,
           W_ww, b_ww, W_wt, b_wt, W_wd, b_wd, W_td, b_td, W_tt, b_tt,
           W_td_cau, W_td_noi, W_tt_cau, W_tt_noi,
           W_td_cau_trans, W_td_noi_trans, W_tt_cau_trans, W_tt_noi_trans):
    ww = _pad_edges(ww_src, ww_dst, ww_w)
    wt = _pad_edges(wt_src, wt_dst, wt_w)
    wd = _pad_edges(wd_src, wd_dst, wd_w)
    td = _pad_edges(td_src, td_dst, td_w)
    tt = _pad_edges(tt_src, tt_dst, tt_w)

    c_ww, c_wt, c_tt, c_wd, c_td = _counts_kernel(
        [ww[1], wt[1], tt[1], wd[1], td[1]],
        [(E_WW, NP_WORD), (E_WT, NP_TOPIC), (E_TT, NP_TOPIC),
         (E_WD, NP_DOC), (E_TD, NP_DOC)])

    tww = _proj_word(feat_word, W_ww, b_ww)
    QW = NP_WORD // 4
    (s_ww,) = _edge_passes(
        [tww], [ww], [NP_WORD],
        [(q // 2, 0, 0, 0, q * QW, QW, E_WW) for q in range(4)],
        QW + 16, 128)

    ttd, ttt = _topic_proj(feat_topic, effect, rand_td, rand_tt,
                           W_td, b_td, W_td_cau, W_td_noi,
                           W_tt, b_tt, W_tt_cau, W_tt_noi,
                           W_td_cau_trans, W_td_noi_trans,
                           W_tt_cau_trans, W_tt_noi_trans)

    h_word, twt, twd = _word_pass2(s_ww, c_ww, W_wt, b_wt, W_wd, b_wd)

    s_wt, s_tt = _edge_passes(
        [twt, ttt], [wt, tt], [NP_TOPIC, NP_TOPIC],
        [(0, 0, 0, 0, 0, NP_TOPIC, E_WT), (1, 1, 1, 1, 0, NP_TOPIC, E_TT)],
        NP_TOPIC + 16, 512)

    HD = NP_DOC // 2
    s_wd, s_td = _edge_passes(
        [twd, ttd], [wd, td], [NP_DOC, NP_DOC],
        [(0, 0, 0, 0, 0, HD, E_WD), (1, 0, 0, 0, HD, HD, E_WD),
         (0, 1, 1, 1, 0, HD, E_TD), (1, 1, 1, 1, HD, HD, E_TD)],
        HD + 16, 256)

    h_topic = _combine(NP_TOPIC, s_wt, c_wt, s_tt, c_tt)
    h_doc = _combine(NP_DOC, s_wd, c_wd, s_td, c_td)

    return (h_word[:N_WORD], h_topic[:N_TOPIC], h_doc[:N_DOC])


# speedup vs baseline: 3.5956x; 1.6640x over previous
import functools
import jax
import jax.numpy as jnp
from jax import lax
from jax.experimental import pallas as pl
from jax.experimental.pallas import tpu as pltpu
from jax.experimental.pallas import tpu_sc as plsc

N_WORD, N_TOPIC, N_DOC = 50000, 5000, 20000
IN_SIZE, OUT_SIZE = 128, 128
B = 512
NTILES = 16
NCORES = 2
D = 128


def _rup(x, m):
    return (x + m - 1) // m * m


NP_WORD = _rup(N_WORD, 1024)
NP_TOPIC = _rup(N_TOPIC, 1024)
NP_DOC = _rup(N_DOC, 1024)

E_WW, E_WT, E_WD, E_TD, E_TT = 300000, 100000, 100000, 50000, 50000


def _proj1_body(x_ref, w_ref, b_ref, o_ref):
    o_ref[...] = jnp.dot(x_ref[...], w_ref[...].T,
                         preferred_element_type=jnp.float32) + b_ref[...]


def _proj_word(x, w, b):
    blk = 400
    return pl.pallas_call(
        _proj1_body,
        grid=(N_WORD // blk,),
        in_specs=[
            pl.BlockSpec((blk, IN_SIZE), lambda i: (i, 0)),
            pl.BlockSpec((OUT_SIZE, IN_SIZE), lambda i: (0, 0)),
            pl.BlockSpec((1, OUT_SIZE), lambda i: (0, 0)),
        ],
        out_specs=pl.BlockSpec((blk, OUT_SIZE), lambda i: (i, 0)),
        out_shape=jax.ShapeDtypeStruct((N_WORD, OUT_SIZE), jnp.float32),
    )(x, w, b.reshape(1, OUT_SIZE))


def _topic_body(x_ref, eff_ref, rtd_ref, rtt_ref,
                wtd_ref, btd_ref, wtdc_ref, wtdn_ref,
                wtt_ref, btt_ref, wttc_ref, wttn_ref,
                ttdc_ref, ttdn_ref, tttc_ref, tttn_ref,
                otd_ref, ott_ref):
    x = x_ref[...]
    eff = eff_ref[...]
    zero = (eff == 0.0).astype(jnp.float32)
    mtd = (rtd_ref[...] < 0.1).astype(jnp.float32) * zero
    mtt = (rtt_ref[...] < 0.1).astype(jnp.float32) * zero

    def trans(m, t_ref):
        t = t_ref[...]
        return (m[:, 0:1] * t[0, 0] + m[:, 1:2] * t[0, 1]
                + m[:, 2:3] * t[0, 2])

    def mm(a, w_ref):
        return jnp.dot(a, w_ref[...].T, preferred_element_type=jnp.float32)

    ew_td = trans(eff, ttdc_ref)
    nw_td = trans(mtd, ttdn_ref)
    ew_tt = trans(eff, tttc_ref)
    nw_tt = trans(mtt, tttn_ref)
    otd_ref[...] = (mm(x, wtd_ref) + btd_ref[...]
                    + mm(x * ew_td, wtdc_ref) - mm(x * nw_td, wtdn_ref))
    ott_ref[...] = (mm(x, wtt_ref) + btt_ref[...]
                    + mm(x * ew_tt, wttc_ref) - mm(x * nw_tt, wttn_ref))


def _topic_proj(x, eff, rtd, rtt, wtd, btd, wtdc, wtdn,
                wtt, btt, wttc, wttn, ttdc, ttdn, tttc, tttn):
    blk = 1000
    mat = lambda: pl.BlockSpec((OUT_SIZE, OUT_SIZE), lambda i: (0, 0))
    tri = lambda: pl.BlockSpec((1, 3), lambda i: (0, 0))
    bia = lambda: pl.BlockSpec((1, OUT_SIZE), lambda i: (0, 0))
    return pl.pallas_call(
        _topic_body,
        grid=(N_TOPIC // blk,),
        in_specs=[
            pl.BlockSpec((blk, OUT_SIZE), lambda i: (i, 0)),
            pl.BlockSpec((blk, 3), lambda i: (i, 0)),
            pl.BlockSpec((blk, 3), lambda i: (i, 0)),
            pl.BlockSpec((blk, 3), lambda i: (i, 0)),
            mat(), bia(), mat(), mat(),
            mat(), bia(), mat(), mat(),
            tri(), tri(), tri(), tri(),
        ],
        out_specs=[pl.BlockSpec((blk, OUT_SIZE), lambda i: (i, 0))] * 2,
        out_shape=[jax.ShapeDtypeStruct((N_TOPIC, OUT_SIZE), jnp.float32)] * 2,
    )(x, eff, rtd, rtt, wtd, btd.reshape(1, -1), wtdc, wtdn,
      wtt, btt.reshape(1, -1), wttc, wttn, ttdc, ttdn, tttc, tttn)


_ONES32 = None


def _cnt_col(c_block):
    ones = jnp.ones((32, 1), jnp.float32)
    return lax.dot_general(c_block, ones, (((0,), (0,)), ((), ())),
                           preferred_element_type=jnp.float32)


def _word2_body(s_ref, c_ref, wwt_ref, bwt_ref, wwd_ref, bwd_ref,
                h_ref, owt_ref, owd_ref):
    c = jnp.maximum(_cnt_col(c_ref[...]), 1.0)
    h = s_ref[...] / c
    h_ref[...] = h
    owt_ref[...] = jnp.dot(h, wwt_ref[...].T,
                           preferred_element_type=jnp.float32) + bwt_ref[...]
    owd_ref[...] = jnp.dot(h, wwd_ref[...].T,
                           preferred_element_type=jnp.float32) + bwd_ref[...]


def _word_pass2(s, cnt, wwt, bwt, wwd, bwd):
    blk = 512
    mat = lambda: pl.BlockSpec((OUT_SIZE, OUT_SIZE), lambda i: (0, 0))
    bia = lambda: pl.BlockSpec((1, OUT_SIZE), lambda i: (0, 0))
    return pl.pallas_call(
        _word2_body,
        grid=(NP_WORD // blk,),
        in_specs=[
            pl.BlockSpec((blk, OUT_SIZE), lambda i: (i, 0)),
            pl.BlockSpec((32, blk), lambda i: (0, i)),
            mat(), bia(), mat(), bia(),
        ],
        out_specs=[pl.BlockSpec((blk, OUT_SIZE), lambda i: (i, 0))] * 3,
        out_shape=[jax.ShapeDtypeStruct((NP_WORD, OUT_SIZE), jnp.float32)] * 3,
    )(s, cnt, wwt, bwt.reshape(1, -1), wwd, bwd.reshape(1, -1))


def _combine_body(sa_ref, ca_ref, sb_ref, cb_ref, o_ref):
    ca = jnp.maximum(_cnt_col(ca_ref[...]), 1.0)
    cb = jnp.maximum(_cnt_col(cb_ref[...]), 1.0)
    o_ref[...] = sa_ref[...] / ca + sb_ref[...] / cb


def _combine(n_pad, sa, ca, sb, cb):
    blk = 512
    return pl.pallas_call(
        _combine_body,
        grid=(n_pad // blk,),
        in_specs=[
            pl.BlockSpec((blk, OUT_SIZE), lambda i: (i, 0)),
            pl.BlockSpec((32, blk), lambda i: (0, i)),
            pl.BlockSpec((blk, OUT_SIZE), lambda i: (i, 0)),
            pl.BlockSpec((32, blk), lambda i: (0, i)),
        ],
        out_specs=pl.BlockSpec((blk, OUT_SIZE), lambda i: (i, 0)),
        out_shape=jax.ShapeDtypeStruct((n_pad, OUT_SIZE), jnp.float32),
    )(sa, ca, sb, cb)


def _mesh():
    return plsc.VectorSubcoreMesh(core_axis_name="c", subcore_axis_name="s")


def _edge_passes(tables, edges, out_shapes, passes, acc_rows, bb):
    nt = len(tables)
    ne = len(edges)
    no = len(out_shapes)

    @functools.partial(
        pl.kernel,
        out_type=[jax.ShapeDtypeStruct((n, D), jnp.float32)
                  for n in out_shapes],
        mesh=_mesh(),
        compiler_params=pltpu.CompilerParams(needs_layout_passes=False),
        scratch_types=[
            pltpu.VMEM_SHARED((acc_rows, D), jnp.float32),
            pltpu.VMEM((bb,), jnp.int32),
            pltpu.VMEM((bb,), jnp.int32),
            pltpu.VMEM((bb,), jnp.float32),
            pltpu.VMEM((bb, D), jnp.float32),
            pltpu.SemaphoreType.DMA,
        ],
    )
    def seg_kernel(*refs):
        tabs = refs[:nt]
        srcs = refs[nt:nt + ne]
        dsts = refs[nt + ne:nt + 2 * ne]
        ws = refs[nt + 2 * ne:nt + 3 * ne]
        outs = refs[nt + 3 * ne:nt + 3 * ne + no]
        acc, src_v, dst_v, w_v, rows_v, sem = refs[nt + 3 * ne + no:]
        cid = lax.axis_index("c")
        sid = lax.axis_index("s")

        for (core, ti, ei, oi, q0, rng, n_e) in passes:
            @pl.when(cid == core)
            def _(ti=ti, ei=ei, oi=oi, q0=q0, rng=rng, n_e=n_e):
                tbl = tabs[ti]
                srch, dsth, wh = srcs[ei], dsts[ei], ws[ei]
                out = outs[oi]
                ep = srch.shape[0]
                nb = ep // (NTILES * bb)
                zrows = rng + 16
                zpt = zrows // NTILES
                rpt = rng // NTILES

                def zr(i, _):
                    for c in range(D // 16):
                        rows_v[i, pl.ds(c * 16, 16)] = (
                            jnp.zeros((16,), jnp.float32))
                    return 0
                lax.fori_loop(0, bb, zr, 0)
                off = 0
                while off < zpt:
                    n = min(bb, zpt - off)
                    pltpu.sync_copy(rows_v.at[pl.ds(0, n)],
                                    acc.at[pl.ds(sid * zpt + off, n)])
                    off += n
                plsc.subcore_barrier()

                def batch(b, _):
                    base = (b * NTILES + sid) * bb

                    @pl.when(base < n_e)
                    def _():
                        pltpu.sync_copy(srch.at[pl.ds(base, bb)], src_v)
                        pltpu.sync_copy(dsth.at[pl.ds(base, bb)], dst_v)
                        pltpu.sync_copy(wh.at[pl.ds(base, bb)], w_v)

                        def remap(g, _):
                            dvec = dst_v[pl.ds(g * 16, 16)] - q0
                            ok = (dvec >= 0) & (dvec < rng)
                            dst_v[pl.ds(g * 16, 16)] = jnp.where(
                                ok, dvec, rng)
                            return 0
                        lax.fori_loop(0, bb // 16, remap, 0)

                        pltpu.async_copy(tbl.at[src_v], rows_v, sem).wait()

                        def scale(g, _):
                            wvec = w_v[pl.ds(g * 16, 16)]
                            for j in range(16):
                                wspl = jnp.broadcast_to(wvec[j], (16,))
                                e = g * 16 + j
                                for c in range(D // 16):
                                    rows_v[e, pl.ds(c * 16, 16)] = (
                                        rows_v[e, pl.ds(c * 16, 16)] * wspl)
                            return 0
                        lax.fori_loop(0, bb // 16, scale, 0)

                        pltpu.sync_copy(rows_v, acc.at[dst_v], add=True)
                    return 0
                lax.fori_loop(0, nb, batch, 0)
                plsc.subcore_barrier()
                pltpu.sync_copy(acc.at[pl.ds(sid * rpt, rpt)],
                                out.at[pl.ds(q0 + sid * rpt, rpt)])
                plsc.subcore_barrier()

    flat = list(tables)
    for i in range(3):
        flat += [e[i] for e in edges]
    res = seg_kernel(*flat)
    return res if isinstance(res, (tuple, list)) else (res,)


def _counts_kernel(dsts_padded, specs):
    ne = len(specs)
    npmax = max(n for (_, n) in specs)

    @functools.partial(
        pl.kernel,
        out_type=[jax.ShapeDtypeStruct((32, n), jnp.float32)
                  for (_, n) in specs],
        mesh=_mesh(),
        compiler_params=pltpu.CompilerParams(needs_layout_passes=False),
        scratch_types=[
            pltpu.VMEM((npmax,), jnp.float32),
            pltpu.VMEM((B,), jnp.int32),
        ],
    )
    def cnt_kernel(*refs):
        dsts = refs[:ne]
        outs = refs[ne:2 * ne]
        plane, dst_v = refs[2 * ne:]
        cid = lax.axis_index("c")
        sid = lax.axis_index("s")
        wid = sid * NCORES + cid
        lane = lax.iota(jnp.int32, 16)
        ones = jnp.ones((16,), jnp.float32)

        for k, (n_e, n_pad) in enumerate(specs):
            dsth = dsts[k]
            out = outs[k]
            ep = dsth.shape[0]
            nb = ep // (32 * B)

            def zp(i, _):
                plane[pl.ds(i * 16, 16)] = jnp.zeros((16,), jnp.float32)
                return 0
            lax.fori_loop(0, n_pad // 16, zp, 0)

            def batch(b, _):
                base = (b * 32 + wid) * B

                @pl.when(base < n_e)
                def _():
                    pltpu.sync_copy(dsth.at[pl.ds(base, B)], dst_v)

                    def grp(g, _):
                        dvec = dst_v[pl.ds(g * 16, 16)]
                        valid = (base + g * 16 + lane) < n_e
                        for l in range(16):
                            m = valid & (lane == l)
                            plsc.addupdate_scatter(
                                plane, [dvec], ones, mask=m)
                        return 0
                    lax.fori_loop(0, B // 16, grp, 0)
                return 0
            lax.fori_loop(0, nb, batch, 0)
            pltpu.sync_copy(plane.at[pl.ds(0, n_pad)], out.at[wid])

    res = cnt_kernel(*dsts_padded)
    return res if isinstance(res, (tuple, list)) else (res,)


WW_EP = 311296
WW_SL = WW_EP // 32
WW_CH = 128
WW_NCHUNK = WW_SL // WW_CH + 1
WW_CAP = WW_NCHUNK * WW_CH
WW_TOT = 4 * 32 * WW_CAP
QW = NP_WORD // 4


def _part_ww(src_p, dst_p, w_p):
    @functools.partial(
        pl.kernel,
        out_type=[jax.ShapeDtypeStruct((WW_TOT,), jnp.int32),
                  jax.ShapeDtypeStruct((WW_TOT,), jnp.int32),
                  jax.ShapeDtypeStruct((WW_TOT,), jnp.float32),
                  jax.ShapeDtypeStruct((512,), jnp.int32)],
        mesh=_mesh(),
        compiler_params=pltpu.CompilerParams(needs_layout_passes=False),
        scratch_types=[
            pltpu.VMEM((512,), jnp.int32),
            pltpu.VMEM((512,), jnp.int32),
            pltpu.VMEM((512,), jnp.float32),
            pltpu.VMEM((4 * 2 * (WW_CH + 16),), jnp.int32),
            pltpu.VMEM((4 * 2 * (WW_CH + 16),), jnp.int32),
            pltpu.VMEM((4 * 2 * (WW_CH + 16),), jnp.float32),
            pltpu.VMEM((16,), jnp.int32),
        ],
    )
    def part(srch, dsth, wh, osrc, odst, ow, ocnt,
             sv, dv, wv, qs, qd, qwb, cv):
        cid = lax.axis_index("c")
        sid = lax.axis_index("s")
        wid = sid * NCORES + cid
        lane = lax.iota(jnp.int32, 16)
        base0 = wid * WW_SL

        def zq(i, _):
            qs[pl.ds(i * 16, 16)] = jnp.zeros((16,), jnp.int32)
            qd[pl.ds(i * 16, 16)] = jnp.zeros((16,), jnp.int32)
            qwb[pl.ds(i * 16, 16)] = jnp.zeros((16,), jnp.float32)
            return 0
        lax.fori_loop(0, 4 * 2 * (WW_CH + 16) // 16, zq, 0)
        HS = WW_CH + 16
        QB = 2 * HS

        def batch(b, carry):
            pltpu.sync_copy(srch.at[pl.ds(base0 + b * 512, 512)], sv)
            pltpu.sync_copy(dsth.at[pl.ds(base0 + b * 512, 512)], dv)
            pltpu.sync_copy(wh.at[pl.ds(base0 + b * 512, 512)], wv)

            def grp(g, c2):
                svec = sv[pl.ds(g * 16, 16)]
                dvec = dv[pl.ds(g * 16, 16)]
                wvec = wv[pl.ds(g * 16, 16)]
                gidx = base0 + b * 512 + g * 16 + lane
                valid = gidx < E_WW
                new = []
                for q in range(4):
                    pos, nf = c2[2 * q], c2[2 * q + 1]
                    m = valid & (dvec >= q * QW) & (dvec < (q + 1) * QW)
                    hb = q * QB + (nf & 1) * HS
                    plsc.store_compressed(
                        qs.at[pl.ds(hb + pos, 16)], svec, mask=m)
                    plsc.store_compressed(
                        qd.at[pl.ds(hb + pos, 16)], dvec, mask=m)
                    plsc.store_compressed(
                        qwb.at[pl.ds(hb + pos, 16)], wvec, mask=m)
                    pos = pos + plsc.all_reduce_population_count(m)[0]
                    full = pos >= WW_CH

                    @pl.when(full)
                    def _(q=q, nf=nf, hb=hb):
                        ob = q * QB + (HS - (nf & 1) * HS)
                        qs[pl.ds(ob, 16)] = qs[pl.ds(hb + WW_CH, 16)]
                        qd[pl.ds(ob, 16)] = qd[pl.ds(hb + WW_CH, 16)]
                        qwb[pl.ds(ob, 16)] = qwb[pl.ds(hb + WW_CH, 16)]
                        dst0 = (q * 32 + wid) * WW_CAP + nf * WW_CH
                        pltpu.sync_copy(qs.at[pl.ds(hb, WW_CH)],
                                        osrc.at[pl.ds(dst0, WW_CH)])
                        pltpu.sync_copy(qd.at[pl.ds(hb, WW_CH)],
                                        odst.at[pl.ds(dst0, WW_CH)])
                        pltpu.sync_copy(qwb.at[pl.ds(hb, WW_CH)],
                                        ow.at[pl.ds(dst0, WW_CH)])

                    pos = jnp.where(full, pos - WW_CH, pos)
                    nf = nf + full.astype(jnp.int32)
                    new += [pos, nf]
                return tuple(new)
            return lax.fori_loop(0, 32, grp, carry)

        z = jnp.int32(0)
        carry = lax.fori_loop(0, WW_SL // 512, batch,
                              (z, z, z, z, z, z, z, z))

        cvec = jnp.zeros((16,), jnp.int32)
        for q in range(4):
            pos, nf = carry[2 * q], carry[2 * q + 1]

            @pl.when(pos > 0)
            def _(q=q, nf=nf):
                hb = q * QB + (nf & 1) * HS
                dst0 = (q * 32 + wid) * WW_CAP + nf * WW_CH
                pltpu.sync_copy(qs.at[pl.ds(hb, WW_CH)],
                                osrc.at[pl.ds(dst0, WW_CH)])
                pltpu.sync_copy(qd.at[pl.ds(hb, WW_CH)],
                                odst.at[pl.ds(dst0, WW_CH)])
                pltpu.sync_copy(qwb.at[pl.ds(hb, WW_CH)],
                                ow.at[pl.ds(dst0, WW_CH)])

            total = nf * WW_CH + pos
            cvec = jnp.where(lane == q, jnp.broadcast_to(total, (16,)), cvec)
        cv[...] = cvec
        pltpu.sync_copy(cv, ocnt.at[pl.ds(wid * 16, 16)])

    return part(src_p, dst_p, w_p)


def _seg_ww(tbl, psrc, pdst, pw, cnts):
    @functools.partial(
        pl.kernel,
        out_type=jax.ShapeDtypeStruct((NP_WORD, D), jnp.float32),
        mesh=_mesh(),
        compiler_params=pltpu.CompilerParams(needs_layout_passes=False),
        scratch_types=[
            pltpu.VMEM_SHARED((QW + 16, D), jnp.float32),
            pltpu.VMEM((WW_CH,), jnp.int32),
            pltpu.VMEM((WW_CH,), jnp.int32),
            pltpu.VMEM((WW_CH,), jnp.float32),
            pltpu.VMEM((WW_CH, D), jnp.float32),
            pltpu.VMEM((512,), jnp.int32),
            pltpu.SemaphoreType.DMA,
        ],
    )
    def seg(tblh, psh, pdh, pwh, ch, out,
            acc, src_v, dst_v, w_v, rows_v, cnt_v, sem):
        cid = lax.axis_index("c")
        sid = lax.axis_index("s")
        lane = lax.iota(jnp.int32, 16)
        pltpu.sync_copy(ch, cnt_v)
        for q in range(4):
            @pl.when(cid == q // 2)
            def _(q=q):
                q0 = q * QW
                zpt = (QW + 16) // NTILES
                rpt = QW // NTILES

                def zr(i, _):
                    for c in range(D // 16):
                        rows_v[i, pl.ds(c * 16, 16)] = (
                            jnp.zeros((16,), jnp.float32))
                    return 0
                lax.fori_loop(0, WW_CH, zr, 0)
                off = 0
                while off < zpt:
                    n = min(WW_CH, zpt - off)
                    pltpu.sync_copy(rows_v.at[pl.ds(0, n)],
                                    acc.at[pl.ds(sid * zpt + off, n)])
                    off += n
                plsc.subcore_barrier()

                for k in range(2):
                    st = sid * 2 + k
                    cnt = cnt_v[pl.ds(st * 16, 16)][q]
                    nch = (cnt + WW_CH - 1) // WW_CH
                    lbase = (q * 32 + st) * WW_CAP

                    def chunk(c, _):
                        cb = lbase + c * WW_CH
                        pltpu.sync_copy(psh.at[pl.ds(cb, WW_CH)], src_v)
                        pltpu.sync_copy(pdh.at[pl.ds(cb, WW_CH)], dst_v)
                        pltpu.sync_copy(pwh.at[pl.ds(cb, WW_CH)], w_v)
                        rem = cnt - c * WW_CH

                        def remap(g, _):
                            dvec = dst_v[pl.ds(g * 16, 16)] - q0
                            ok = (((g * 16 + lane) < rem)
                                  & (dvec >= 0) & (dvec < QW))
                            dst_v[pl.ds(g * 16, 16)] = jnp.where(
                                ok, dvec, QW)
                            return 0
                        lax.fori_loop(0, WW_CH // 16, remap, 0)

                        pltpu.async_copy(tblh.at[src_v], rows_v, sem).wait()

                        def scale(g, _):
                            wvec = w_v[pl.ds(g * 16, 16)]
                            for j in range(16):
                                wspl = jnp.broadcast_to(wvec[j], (16,))
                                e = g * 16 + j
                                for c2 in range(D // 16):
                                    rows_v[e, pl.ds(c2 * 16, 16)] = (
                                        rows_v[e, pl.ds(c2 * 16, 16)] * wspl)
                            return 0
                        lax.fori_loop(0, WW_CH // 16, scale, 0)

                        pltpu.sync_copy(rows_v, acc.at[dst_v], add=True)
                        return 0
                    lax.fori_loop(0, nch, chunk, 0)
                plsc.subcore_barrier()
                pltpu.sync_copy(acc.at[pl.ds(sid * rpt, rpt)],
                                out.at[pl.ds(q0 + sid * rpt, rpt)])
                plsc.subcore_barrier()

    return seg(tbl, psrc, pdst, pw, cnts)


def _pad_edges(src, dst, w):
    e = src.shape[0]
    ep = _rup(e, 32 * B)
    pad = ep - e
    src_p = jnp.pad(src.astype(jnp.int32), (0, pad))
    dst_p = jnp.pad(dst.astype(jnp.int32), (0, pad))
    w_p = jnp.pad(w, (0, pad))
    return src_p, dst_p, w_p


def kernel(feat_word, feat_topic, effect,
           ww_src, ww_dst, ww_w, wt_src, wt_dst, wt_w,
           wd_src, wd_dst, wd_w, td_src, td_dst, td_w,
           tt_src, tt_dst, tt_w, rand_td, rand_tt,
           W_ww, b_ww, W_wt, b_wt, W_wd, b_wd, W_td, b_td, W_tt, b_tt,
           W_td_cau, W_td_noi, W_tt_cau, W_tt_noi,
           W_td_cau_trans, W_td_noi_trans, W_tt_cau_trans, W_tt_noi_trans):
    ww = _pad_edges(ww_src, ww_dst, ww_w)
    wt = _pad_edges(wt_src, wt_dst, wt_w)
    wd = _pad_edges(wd_src, wd_dst, wd_w)
    td = _pad_edges(td_src, td_dst, td_w)
    tt = _pad_edges(tt_src, tt_dst, tt_w)

    c_ww, c_wt, c_tt, c_wd, c_td = _counts_kernel(
        [ww[1], wt[1], tt[1], wd[1], td[1]],
        [(E_WW, NP_WORD), (E_WT, NP_TOPIC), (E_TT, NP_TOPIC),
         (E_WD, NP_DOC), (E_TD, NP_DOC)])

    tww = _proj_word(feat_word, W_ww, b_ww)
    pws, pwd, pww, wcnt = _part_ww(*ww)
    s_ww = _seg_ww(tww, pws, pwd, pww, wcnt)

    ttd, ttt = _topic_proj(feat_topic, effect, rand_td, rand_tt,
                           W_td, b_td, W_td_cau, W_td_noi,
                           W_tt, b_tt, W_tt_cau, W_tt_noi,
                           W_td_cau_trans, W_td_noi_trans,
                           W_tt_cau_trans, W_tt_noi_trans)

    h_word, twt, twd = _word_pass2(s_ww, c_ww, W_wt, b_wt, W_wd, b_wd)

    s_wt, s_tt = _edge_passes(
        [twt, ttt], [wt, tt], [NP_TOPIC, NP_TOPIC],
        [(0, 0, 0, 0, 0, NP_TOPIC, E_WT), (1, 1, 1, 1, 0, NP_TOPIC, E_TT)],
        NP_TOPIC + 16, 512)

    HD = NP_DOC // 2
    s_wd, s_td = _edge_passes(
        [twd, ttd], [wd, td], [NP_DOC, NP_DOC],
        [(0, 0, 0, 0, 0, HD, E_WD), (1, 0, 0, 0, HD, HD, E_WD),
         (0, 1, 1, 1, 0, HD, E_TD), (1, 1, 1, 1, HD, HD, E_TD)],
        HD + 16, 256)

    h_topic = _combine(NP_TOPIC, s_wt, c_wt, s_tt, c_tt)
    h_doc = _combine(NP_DOC, s_wd, c_wd, s_td, c_td)

    return (h_word[:N_WORD], h_topic[:N_TOPIC], h_doc[:N_DOC])


# speedup vs baseline: 4.0072x; 1.1144x over previous
import functools
import jax
import jax.numpy as jnp
from jax import lax
from jax.experimental import pallas as pl
from jax.experimental.pallas import tpu as pltpu
from jax.experimental.pallas import tpu_sc as plsc

N_WORD, N_TOPIC, N_DOC = 50000, 5000, 20000
IN_SIZE, OUT_SIZE = 128, 128
B = 512
NTILES = 16
NCORES = 2
D = 128


def _rup(x, m):
    return (x + m - 1) // m * m


NP_WORD = _rup(N_WORD, 1024)
NP_TOPIC = _rup(N_TOPIC, 1024)
NP_DOC = _rup(N_DOC, 1024)

E_WW, E_WT, E_WD, E_TD, E_TT = 300000, 100000, 100000, 50000, 50000


def _proj1_body(x_ref, w_ref, b_ref, o_ref):
    o_ref[...] = jnp.dot(x_ref[...], w_ref[...].T,
                         preferred_element_type=jnp.float32) + b_ref[...]


def _proj_word(x, w, b):
    blk = 400
    return pl.pallas_call(
        _proj1_body,
        grid=(N_WORD // blk,),
        in_specs=[
            pl.BlockSpec((blk, IN_SIZE), lambda i: (i, 0)),
            pl.BlockSpec((OUT_SIZE, IN_SIZE), lambda i: (0, 0)),
            pl.BlockSpec((1, OUT_SIZE), lambda i: (0, 0)),
        ],
        out_specs=pl.BlockSpec((blk, OUT_SIZE), lambda i: (i, 0)),
        out_shape=jax.ShapeDtypeStruct((N_WORD, OUT_SIZE), jnp.float32),
    )(x, w, b.reshape(1, OUT_SIZE))


def _topic_body(x_ref, eff_ref, rtd_ref, rtt_ref,
                wtd_ref, btd_ref, wtdc_ref, wtdn_ref,
                wtt_ref, btt_ref, wttc_ref, wttn_ref,
                ttdc_ref, ttdn_ref, tttc_ref, tttn_ref,
                otd_ref, ott_ref):
    x = x_ref[...]
    eff = eff_ref[...]
    zero = (eff == 0.0).astype(jnp.float32)
    mtd = (rtd_ref[...] < 0.1).astype(jnp.float32) * zero
    mtt = (rtt_ref[...] < 0.1).astype(jnp.float32) * zero

    def trans(m, t_ref):
        t = t_ref[...]
        return (m[:, 0:1] * t[0, 0] + m[:, 1:2] * t[0, 1]
                + m[:, 2:3] * t[0, 2])

    def mm(a, w_ref):
        return jnp.dot(a, w_ref[...].T, preferred_element_type=jnp.float32)

    ew_td = trans(eff, ttdc_ref)
    nw_td = trans(mtd, ttdn_ref)
    ew_tt = trans(eff, tttc_ref)
    nw_tt = trans(mtt, tttn_ref)
    otd_ref[...] = (mm(x, wtd_ref) + btd_ref[...]
                    + mm(x * ew_td, wtdc_ref) - mm(x * nw_td, wtdn_ref))
    ott_ref[...] = (mm(x, wtt_ref) + btt_ref[...]
                    + mm(x * ew_tt, wttc_ref) - mm(x * nw_tt, wttn_ref))


def _topic_proj(x, eff, rtd, rtt, wtd, btd, wtdc, wtdn,
                wtt, btt, wttc, wttn, ttdc, ttdn, tttc, tttn):
    blk = 1000
    mat = lambda: pl.BlockSpec((OUT_SIZE, OUT_SIZE), lambda i: (0, 0))
    tri = lambda: pl.BlockSpec((1, 3), lambda i: (0, 0))
    bia = lambda: pl.BlockSpec((1, OUT_SIZE), lambda i: (0, 0))
    return pl.pallas_call(
        _topic_body,
        grid=(N_TOPIC // blk,),
        in_specs=[
            pl.BlockSpec((blk, OUT_SIZE), lambda i: (i, 0)),
            pl.BlockSpec((blk, 3), lambda i: (i, 0)),
            pl.BlockSpec((blk, 3), lambda i: (i, 0)),
            pl.BlockSpec((blk, 3), lambda i: (i, 0)),
            mat(), bia(), mat(), mat(),
            mat(), bia(), mat(), mat(),
            tri(), tri(), tri(), tri(),
        ],
        out_specs=[pl.BlockSpec((blk, OUT_SIZE), lambda i: (i, 0))] * 2,
        out_shape=[jax.ShapeDtypeStruct((N_TOPIC, OUT_SIZE), jnp.float32)] * 2,
    )(x, eff, rtd, rtt, wtd, btd.reshape(1, -1), wtdc, wtdn,
      wtt, btt.reshape(1, -1), wttc, wttn, ttdc, ttdn, tttc, tttn)


_ONES32 = None


def _cnt_col(c_block):
    ones = jnp.ones((32, 1), jnp.float32)
    return lax.dot_general(c_block, ones, (((0,), (0,)), ((), ())),
                           preferred_element_type=jnp.float32)


def _word2_body(s_ref, c_ref, wwt_ref, bwt_ref, wwd_ref, bwd_ref,
                h_ref, owt_ref, owd_ref):
    c = jnp.maximum(_cnt_col(c_ref[...]), 1.0)
    h = s_ref[...] / c
    h_ref[...] = h
    owt_ref[...] = jnp.dot(h, wwt_ref[...].T,
                           preferred_element_type=jnp.float32) + bwt_ref[...]
    owd_ref[...] = jnp.dot(h, wwd_ref[...].T,
                           preferred_element_type=jnp.float32) + bwd_ref[...]


def _word_pass2(s, cnt, wwt, bwt, wwd, bwd):
    blk = 512
    mat = lambda: pl.BlockSpec((OUT_SIZE, OUT_SIZE), lambda i: (0, 0))
    bia = lambda: pl.BlockSpec((1, OUT_SIZE), lambda i: (0, 0))
    return pl.pallas_call(
        _word2_body,
        grid=(NP_WORD // blk,),
        in_specs=[
            pl.BlockSpec((blk, OUT_SIZE), lambda i: (i, 0)),
            pl.BlockSpec((32, blk), lambda i: (0, i)),
            mat(), bia(), mat(), bia(),
        ],
        out_specs=[pl.BlockSpec((blk, OUT_SIZE), lambda i: (i, 0))] * 3,
        out_shape=[jax.ShapeDtypeStruct((NP_WORD, OUT_SIZE), jnp.float32)] * 3,
    )(s, cnt, wwt, bwt.reshape(1, -1), wwd, bwd.reshape(1, -1))


def _combine2_body(sa0_ref, sa1_ref, ca_ref, sb0_ref, sb1_ref, cb_ref,
                   o_ref):
    ca = jnp.maximum(_cnt_col(ca_ref[...]), 1.0)
    cb = jnp.maximum(_cnt_col(cb_ref[...]), 1.0)
    o_ref[...] = ((sa0_ref[...] + sa1_ref[...]) / ca
                  + (sb0_ref[...] + sb1_ref[...]) / cb)


def _combine2(n_pad, sa0, sa1, ca, sb0, sb1, cb):
    blk = 512
    sb_spec = lambda: pl.BlockSpec((blk, OUT_SIZE), lambda i: (i, 0))
    cb_spec = lambda: pl.BlockSpec((32, blk), lambda i: (0, i))
    return pl.pallas_call(
        _combine2_body,
        grid=(n_pad // blk,),
        in_specs=[sb_spec(), sb_spec(), cb_spec(),
                  sb_spec(), sb_spec(), cb_spec()],
        out_specs=pl.BlockSpec((blk, OUT_SIZE), lambda i: (i, 0)),
        out_shape=jax.ShapeDtypeStruct((n_pad, OUT_SIZE), jnp.float32),
    )(sa0, sa1, ca, sb0, sb1, cb)


def _combine_body(sa_ref, ca_ref, sb_ref, cb_ref, o_ref):
    ca = jnp.maximum(_cnt_col(ca_ref[...]), 1.0)
    cb = jnp.maximum(_cnt_col(cb_ref[...]), 1.0)
    o_ref[...] = sa_ref[...] / ca + sb_ref[...] / cb


def _combine(n_pad, sa, ca, sb, cb):
    blk = 512
    return pl.pallas_call(
        _combine_body,
        grid=(n_pad // blk,),
        in_specs=[
            pl.BlockSpec((blk, OUT_SIZE), lambda i: (i, 0)),
            pl.BlockSpec((32, blk), lambda i: (0, i)),
            pl.BlockSpec((blk, OUT_SIZE), lambda i: (i, 0)),
            pl.BlockSpec((32, blk), lambda i: (0, i)),
        ],
        out_specs=pl.BlockSpec((blk, OUT_SIZE), lambda i: (i, 0)),
        out_shape=jax.ShapeDtypeStruct((n_pad, OUT_SIZE), jnp.float32),
    )(sa, ca, sb, cb)


def _mesh():
    return plsc.VectorSubcoreMesh(core_axis_name="c", subcore_axis_name="s")


def _edge_passes(tables, edges, out_shapes, passes, acc_rows, bb):
    nt = len(tables)
    ne = len(edges)
    no = len(out_shapes)

    @functools.partial(
        pl.kernel,
        out_type=[jax.ShapeDtypeStruct((n, D), jnp.float32)
                  for n in out_shapes],
        mesh=_mesh(),
        compiler_params=pltpu.CompilerParams(needs_layout_passes=False),
        scratch_types=[
            pltpu.VMEM_SHARED((acc_rows, D), jnp.float32),
            pltpu.VMEM((bb,), jnp.int32),
            pltpu.VMEM((bb,), jnp.int32),
            pltpu.VMEM((bb,), jnp.float32),
            pltpu.VMEM((bb, D), jnp.float32),
            pltpu.SemaphoreType.DMA,
            pltpu.SemaphoreType.DMA,
        ],
    )
    def seg_kernel(*refs):
        tabs = refs[:nt]
        srcs = refs[nt:nt + ne]
        dsts = refs[nt + ne:nt + 2 * ne]
        ws = refs[nt + 2 * ne:nt + 3 * ne]
        outs = refs[nt + 3 * ne:nt + 3 * ne + no]
        acc, src_v, dst_v, w_v, rows_v, sem, sem_i = refs[nt + 3 * ne + no:]
        cid = lax.axis_index("c")
        sid = lax.axis_index("s")

        for (core, ti, ei, oi, q0, rng, n_e, e0, e1) in passes:
            @pl.when(cid == core)
            def _(ti=ti, ei=ei, oi=oi, q0=q0, rng=rng, n_e=n_e,
                  e0=e0, e1=e1):
                tbl = tabs[ti]
                srch, dsth, wh = srcs[ei], dsts[ei], ws[ei]
                out = outs[oi]
                nb = (e1 - e0) // (NTILES * bb)
                zrows = rng + 16
                zpt = zrows // NTILES
                rpt = rng // NTILES

                def zr(i, _):
                    for c in range(D // 16):
                        rows_v[i, pl.ds(c * 16, 16)] = (
                            jnp.zeros((16,), jnp.float32))
                    return 0
                lax.fori_loop(0, bb, zr, 0)
                off = 0
                while off < zpt:
                    n = min(bb, zpt - off)
                    pltpu.sync_copy(rows_v.at[pl.ds(0, n)],
                                    acc.at[pl.ds(sid * zpt + off, n)])
                    off += n
                plsc.subcore_barrier()

                def batch(b, _):
                    base = e0 + (b * NTILES + sid) * bb

                    @pl.when(base < n_e)
                    def _():
                        d1 = pltpu.async_copy(
                            srch.at[pl.ds(base, bb)], src_v, sem_i)
                        d2 = pltpu.async_copy(
                            dsth.at[pl.ds(base, bb)], dst_v, sem_i)
                        d3 = pltpu.async_copy(
                            wh.at[pl.ds(base, bb)], w_v, sem_i)
                        d1.wait()
                        g2 = pltpu.async_copy(tbl.at[src_v], rows_v, sem)
                        d2.wait()

                        def remap(g, _):
                            dvec = dst_v[pl.ds(g * 16, 16)] - q0
                            ok = (dvec >= 0) & (dvec < rng)
                            dst_v[pl.ds(g * 16, 16)] = jnp.where(
                                ok, dvec, rng)
                            return 0
                        lax.fori_loop(0, bb // 16, remap, 0)
                        d3.wait()
                        g2.wait()

                        def scale(g, _):
                            wvec = w_v[pl.ds(g * 16, 16)]
                            for j in range(16):
                                wspl = jnp.broadcast_to(wvec[j], (16,))
                                e = g * 16 + j
                                for c in range(D // 16):
                                    rows_v[e, pl.ds(c * 16, 16)] = (
                                        rows_v[e, pl.ds(c * 16, 16)] * wspl)
                            return 0
                        lax.fori_loop(0, bb // 16, scale, 0)

                        pltpu.sync_copy(rows_v, acc.at[dst_v], add=True)
                    return 0
                lax.fori_loop(0, nb, batch, 0)
                plsc.subcore_barrier()
                pltpu.sync_copy(acc.at[pl.ds(sid * rpt, rpt)],
                                out.at[pl.ds(q0 + sid * rpt, rpt)])
                plsc.subcore_barrier()

    flat = list(tables)
    for i in range(3):
        flat += [e[i] for e in edges]
    res = seg_kernel(*flat)
    return res if isinstance(res, (tuple, list)) else (res,)


def _counts_kernel(dsts_padded, specs):
    ne = len(specs)
    npmax = max(n for (_, n) in specs)

    @functools.partial(
        pl.kernel,
        out_type=[jax.ShapeDtypeStruct((32, n), jnp.float32)
                  for (_, n) in specs],
        mesh=_mesh(),
        compiler_params=pltpu.CompilerParams(needs_layout_passes=False),
        scratch_types=[
            pltpu.VMEM((npmax,), jnp.float32),
            pltpu.VMEM((B,), jnp.int32),
        ],
    )
    def cnt_kernel(*refs):
        dsts = refs[:ne]
        outs = refs[ne:2 * ne]
        plane, dst_v = refs[2 * ne:]
        cid = lax.axis_index("c")
        sid = lax.axis_index("s")
        wid = sid * NCORES + cid
        lane = lax.iota(jnp.int32, 16)
        ones = jnp.ones((16,), jnp.float32)

        for k, (n_e, n_pad) in enumerate(specs):
            dsth = dsts[k]
            out = outs[k]
            ep = dsth.shape[0]
            nb = ep // (32 * B)

            def zp(i, _):
                plane[pl.ds(i * 16, 16)] = jnp.zeros((16,), jnp.float32)
                return 0
            lax.fori_loop(0, n_pad // 16, zp, 0)

            def batch(b, _):
                base = (b * 32 + wid) * B

                @pl.when(base < n_e)
                def _():
                    pltpu.sync_copy(dsth.at[pl.ds(base, B)], dst_v)

                    def grp(g, _):
                        dvec = dst_v[pl.ds(g * 16, 16)]
                        valid = (base + g * 16 + lane) < n_e
                        for l in range(16):
                            m = valid & (lane == l)
                            plsc.addupdate_scatter(
                                plane, [dvec], ones, mask=m)
                        return 0
                    lax.fori_loop(0, B // 16, grp, 0)
                return 0
            lax.fori_loop(0, nb, batch, 0)
            pltpu.sync_copy(plane.at[pl.ds(0, n_pad)], out.at[wid])

    res = cnt_kernel(*dsts_padded)
    return res if isinstance(res, (tuple, list)) else (res,)


WW_EP = 311296
WW_SL = WW_EP // 32
WW_CH = 128
WW_NCHUNK = WW_SL // WW_CH + 1
WW_CAP = WW_NCHUNK * WW_CH
WW_TOT = 4 * 32 * WW_CAP
QW = NP_WORD // 4


def _part_ww(src_p, dst_p, w_p):
    @functools.partial(
        pl.kernel,
        out_type=[jax.ShapeDtypeStruct((WW_TOT,), jnp.int32),
                  jax.ShapeDtypeStruct((WW_TOT,), jnp.int32),
                  jax.ShapeDtypeStruct((WW_TOT,), jnp.float32),
                  jax.ShapeDtypeStruct((512,), jnp.int32)],
        mesh=_mesh(),
        compiler_params=pltpu.CompilerParams(needs_layout_passes=False),
        scratch_types=[
            pltpu.VMEM((512,), jnp.int32),
            pltpu.VMEM((512,), jnp.int32),
            pltpu.VMEM((512,), jnp.float32),
            pltpu.VMEM((4 * 2 * (WW_CH + 16),), jnp.int32),
            pltpu.VMEM((4 * 2 * (WW_CH + 16),), jnp.int32),
            pltpu.VMEM((4 * 2 * (WW_CH + 16),), jnp.float32),
            pltpu.VMEM((16,), jnp.int32),
        ],
    )
    def part(srch, dsth, wh, osrc, odst, ow, ocnt,
             sv, dv, wv, qs, qd, qwb, cv):
        cid = lax.axis_index("c")
        sid = lax.axis_index("s")
        wid = sid * NCORES + cid
        lane = lax.iota(jnp.int32, 16)
        base0 = wid * WW_SL

        def zq(i, _):
            qs[pl.ds(i * 16, 16)] = jnp.zeros((16,), jnp.int32)
            qd[pl.ds(i * 16, 16)] = jnp.zeros((16,), jnp.int32)
            qwb[pl.ds(i * 16, 16)] = jnp.zeros((16,), jnp.float32)
            return 0
        lax.fori_loop(0, 4 * 2 * (WW_CH + 16) // 16, zq, 0)
        HS = WW_CH + 16
        QB = 2 * HS

        def batch(b, carry):
            pltpu.sync_copy(srch.at[pl.ds(base0 + b * 512, 512)], sv)
            pltpu.sync_copy(dsth.at[pl.ds(base0 + b * 512, 512)], dv)
            pltpu.sync_copy(wh.at[pl.ds(base0 + b * 512, 512)], wv)

            def grp(g, c2):
                svec = sv[pl.ds(g * 16, 16)]
                dvec = dv[pl.ds(g * 16, 16)]
                wvec = wv[pl.ds(g * 16, 16)]
                gidx = base0 + b * 512 + g * 16 + lane
                valid = gidx < E_WW
                new = []
                for q in range(4):
                    pos, nf = c2[2 * q], c2[2 * q + 1]
                    m = valid & (dvec >= q * QW) & (dvec < (q + 1) * QW)
                    hb = q * QB + (nf & 1) * HS
                    plsc.store_compressed(
                        qs.at[pl.ds(hb + pos, 16)], svec, mask=m)
                    plsc.store_compressed(
                        qd.at[pl.ds(hb + pos, 16)], dvec, mask=m)
                    plsc.store_compressed(
                        qwb.at[pl.ds(hb + pos, 16)], wvec, mask=m)
                    pos = pos + plsc.all_reduce_population_count(m)[0]
                    full = pos >= WW_CH

                    @pl.when(full)
                    def _(q=q, nf=nf, hb=hb):
                        ob = q * QB + (HS - (nf & 1) * HS)
                        qs[pl.ds(ob, 16)] = qs[pl.ds(hb + WW_CH, 16)]
                        qd[pl.ds(ob, 16)] = qd[pl.ds(hb + WW_CH, 16)]
                        qwb[pl.ds(ob, 16)] = qwb[pl.ds(hb + WW_CH, 16)]
                        dst0 = (q * 32 + wid) * WW_CAP + nf * WW_CH
                        pltpu.sync_copy(qs.at[pl.ds(hb, WW_CH)],
                                        osrc.at[pl.ds(dst0, WW_CH)])
                        pltpu.sync_copy(qd.at[pl.ds(hb, WW_CH)],
                                        odst.at[pl.ds(dst0, WW_CH)])
                        pltpu.sync_copy(qwb.at[pl.ds(hb, WW_CH)],
                                        ow.at[pl.ds(dst0, WW_CH)])

                    pos = jnp.where(full, pos - WW_CH, pos)
                    nf = nf + full.astype(jnp.int32)
                    new += [pos, nf]
                return tuple(new)
            return lax.fori_loop(0, 32, grp, carry)

        z = jnp.int32(0)
        carry = lax.fori_loop(0, WW_SL // 512, batch,
                              (z, z, z, z, z, z, z, z))

        cvec = jnp.zeros((16,), jnp.int32)
        for q in range(4):
            pos, nf = carry[2 * q], carry[2 * q + 1]

            @pl.when(pos > 0)
            def _(q=q, nf=nf):
                hb = q * QB + (nf & 1) * HS
                dst0 = (q * 32 + wid) * WW_CAP + nf * WW_CH
                pltpu.sync_copy(qs.at[pl.ds(hb, WW_CH)],
                                osrc.at[pl.ds(dst0, WW_CH)])
                pltpu.sync_copy(qd.at[pl.ds(hb, WW_CH)],
                                odst.at[pl.ds(dst0, WW_CH)])
                pltpu.sync_copy(qwb.at[pl.ds(hb, WW_CH)],
                                ow.at[pl.ds(dst0, WW_CH)])

            total = nf * WW_CH + pos
            cvec = jnp.where(lane == q, jnp.broadcast_to(total, (16,)), cvec)
        cv[...] = cvec
        pltpu.sync_copy(cv, ocnt.at[pl.ds(wid * 16, 16)])

    return part(src_p, dst_p, w_p)


def _seg_ww(tbl, psrc, pdst, pw, cnts):
    @functools.partial(
        pl.kernel,
        out_type=jax.ShapeDtypeStruct((NP_WORD, D), jnp.float32),
        mesh=_mesh(),
        compiler_params=pltpu.CompilerParams(needs_layout_passes=False),
        scratch_types=[
            pltpu.VMEM_SHARED((QW + 16, D), jnp.float32),
            pltpu.VMEM((WW_CH,), jnp.int32),
            pltpu.VMEM((WW_CH,), jnp.int32),
            pltpu.VMEM((WW_CH,), jnp.float32),
            pltpu.VMEM((WW_CH, D), jnp.float32),
            pltpu.VMEM((512,), jnp.int32),
            pltpu.SemaphoreType.DMA,
            pltpu.SemaphoreType.DMA,
        ],
    )
    def seg(tblh, psh, pdh, pwh, ch, out,
            acc, src_v, dst_v, w_v, rows_v, cnt_v, sem, sem_i):
        cid = lax.axis_index("c")
        sid = lax.axis_index("s")
        lane = lax.iota(jnp.int32, 16)
        pltpu.sync_copy(ch, cnt_v)
        for q in range(4):
            @pl.when(cid == q // 2)
            def _(q=q):
                q0 = q * QW
                zpt = (QW + 16) // NTILES
                rpt = QW // NTILES

                def zr(i, _):
                    for c in range(D // 16):
                        rows_v[i, pl.ds(c * 16, 16)] = (
                            jnp.zeros((16,), jnp.float32))
                    return 0
                lax.fori_loop(0, WW_CH, zr, 0)
                off = 0
                while off < zpt:
                    n = min(WW_CH, zpt - off)
                    pltpu.sync_copy(rows_v.at[pl.ds(0, n)],
                                    acc.at[pl.ds(sid * zpt + off, n)])
                    off += n
                plsc.subcore_barrier()

                for k in range(2):
                    st = sid * 2 + k
                    cnt = cnt_v[pl.ds(st * 16, 16)][q]
                    nch = (cnt + WW_CH - 1) // WW_CH
                    lbase = (q * 32 + st) * WW_CAP

                    def chunk(c, _):
                        cb = lbase + c * WW_CH
                        d1 = pltpu.async_copy(
                            psh.at[pl.ds(cb, WW_CH)], src_v, sem_i)
                        d2 = pltpu.async_copy(
                            pdh.at[pl.ds(cb, WW_CH)], dst_v, sem_i)
                        d3 = pltpu.async_copy(
                            pwh.at[pl.ds(cb, WW_CH)], w_v, sem_i)
                        rem = cnt - c * WW_CH
                        d1.wait()
                        g = pltpu.async_copy(tblh.at[src_v], rows_v, sem)
                        d2.wait()

                        def remap(gi, _):
                            dvec = dst_v[pl.ds(gi * 16, 16)] - q0
                            ok = (((gi * 16 + lane) < rem)
                                  & (dvec >= 0) & (dvec < QW))
                            dst_v[pl.ds(gi * 16, 16)] = jnp.where(
                                ok, dvec, QW)
                            return 0
                        lax.fori_loop(0, WW_CH // 16, remap, 0)
                        d3.wait()
                        g.wait()

                        def scale(g, _):
                            wvec = w_v[pl.ds(g * 16, 16)]
                            for j in range(16):
                                wspl = jnp.broadcast_to(wvec[j], (16,))
                                e = g * 16 + j
                                for c2 in range(D // 16):
                                    rows_v[e, pl.ds(c2 * 16, 16)] = (
                                        rows_v[e, pl.ds(c2 * 16, 16)] * wspl)
                            return 0
                        lax.fori_loop(0, WW_CH // 16, scale, 0)

                        pltpu.sync_copy(rows_v, acc.at[dst_v], add=True)
                        return 0
                    lax.fori_loop(0, nch, chunk, 0)
                plsc.subcore_barrier()
                pltpu.sync_copy(acc.at[pl.ds(sid * rpt, rpt)],
                                out.at[pl.ds(q0 + sid * rpt, rpt)])
                plsc.subcore_barrier()

    return seg(tbl, psrc, pdst, pw, cnts)


def _pad_edges(src, dst, w):
    e = src.shape[0]
    ep = _rup(e, 32 * B)
    pad = ep - e
    src_p = jnp.pad(src.astype(jnp.int32), (0, pad))
    dst_p = jnp.pad(dst.astype(jnp.int32), (0, pad))
    w_p = jnp.pad(w, (0, pad))
    return src_p, dst_p, w_p


def kernel(feat_word, feat_topic, effect,
           ww_src, ww_dst, ww_w, wt_src, wt_dst, wt_w,
           wd_src, wd_dst, wd_w, td_src, td_dst, td_w,
           tt_src, tt_dst, tt_w, rand_td, rand_tt,
           W_ww, b_ww, W_wt, b_wt, W_wd, b_wd, W_td, b_td, W_tt, b_tt,
           W_td_cau, W_td_noi, W_tt_cau, W_tt_noi,
           W_td_cau_trans, W_td_noi_trans, W_tt_cau_trans, W_tt_noi_trans):
    ww = _pad_edges(ww_src, ww_dst, ww_w)
    wt = _pad_edges(wt_src, wt_dst, wt_w)
    wd = _pad_edges(wd_src, wd_dst, wd_w)
    td = _pad_edges(td_src, td_dst, td_w)
    tt = _pad_edges(tt_src, tt_dst, tt_w)

    c_ww, c_wt, c_tt, c_wd, c_td = _counts_kernel(
        [ww[1], wt[1], tt[1], wd[1], td[1]],
        [(E_WW, NP_WORD), (E_WT, NP_TOPIC), (E_TT, NP_TOPIC),
         (E_WD, NP_DOC), (E_TD, NP_DOC)])

    tww = _proj_word(feat_word, W_ww, b_ww)
    pws, pwd, pww, wcnt = _part_ww(*ww)
    s_ww = _seg_ww(tww, pws, pwd, pww, wcnt)

    ttd, ttt = _topic_proj(feat_topic, effect, rand_td, rand_tt,
                           W_td, b_td, W_td_cau, W_td_noi,
                           W_tt, b_tt, W_tt_cau, W_tt_noi,
                           W_td_cau_trans, W_td_noi_trans,
                           W_tt_cau_trans, W_tt_noi_trans)

    h_word, twt, twd = _word_pass2(s_ww, c_ww, W_wt, b_wt, W_wd, b_wd)

    wt_h = wt[0].shape[0] // 2
    tt_h = tt[0].shape[0] // 2
    s_wt0, s_wt1, s_tt0, s_tt1 = _edge_passes(
        [twt, ttt], [wt, tt], [NP_TOPIC] * 4,
        [(0, 0, 0, 0, 0, NP_TOPIC, E_WT, 0, wt_h),
         (1, 0, 0, 1, 0, NP_TOPIC, E_WT, wt_h, 2 * wt_h),
         (0, 1, 1, 2, 0, NP_TOPIC, E_TT, 0, tt_h),
         (1, 1, 1, 3, 0, NP_TOPIC, E_TT, tt_h, 2 * tt_h)],
        NP_TOPIC + 16, 512)

    HD = NP_DOC // 2
    wd_e = wd[0].shape[0]
    td_e = td[0].shape[0]
    s_wd, s_td = _edge_passes(
        [twd, ttd], [wd, td], [NP_DOC, NP_DOC],
        [(0, 0, 0, 0, 0, HD, E_WD, 0, wd_e), (1, 0, 0, 0, HD, HD, E_WD, 0, wd_e),
         (0, 1, 1, 1, 0, HD, E_TD, 0, td_e), (1, 1, 1, 1, HD, HD, E_TD, 0, td_e)],
        HD + 16, 256)

    h_topic = _combine2(NP_TOPIC, s_wt0, s_wt1, c_wt, s_tt0, s_tt1, c_tt)
    h_doc = _combine(NP_DOC, s_wd, c_wd, s_td, c_td)

    return (h_word[:N_WORD], h_topic[:N_TOPIC], h_doc[:N_DOC])


# speedup vs baseline: 4.0701x; 1.0157x over previous
import functools
import jax
import jax.numpy as jnp
from jax import lax
from jax.experimental import pallas as pl
from jax.experimental.pallas import tpu as pltpu
from jax.experimental.pallas import tpu_sc as plsc

N_WORD, N_TOPIC, N_DOC = 50000, 5000, 20000
IN_SIZE, OUT_SIZE = 128, 128
B = 512
NTILES = 16
NCORES = 2
D = 128


def _rup(x, m):
    return (x + m - 1) // m * m


NP_WORD = _rup(N_WORD, 1024)
NP_TOPIC = _rup(N_TOPIC, 1024)
NP_DOC = _rup(N_DOC, 1024)

E_WW, E_WT, E_WD, E_TD, E_TT = 300000, 100000, 100000, 50000, 50000


def _proj1_body(x_ref, w_ref, b_ref, o_ref):
    o_ref[...] = jnp.dot(x_ref[...], w_ref[...].T,
                         preferred_element_type=jnp.float32) + b_ref[...]


def _proj_word(x, w, b):
    blk = 400
    return pl.pallas_call(
        _proj1_body,
        grid=(N_WORD // blk,),
        in_specs=[
            pl.BlockSpec((blk, IN_SIZE), lambda i: (i, 0)),
            pl.BlockSpec((OUT_SIZE, IN_SIZE), lambda i: (0, 0)),
            pl.BlockSpec((1, OUT_SIZE), lambda i: (0, 0)),
        ],
        out_specs=pl.BlockSpec((blk, OUT_SIZE), lambda i: (i, 0)),
        out_shape=jax.ShapeDtypeStruct((N_WORD, OUT_SIZE), jnp.float32),
    )(x, w, b.reshape(1, OUT_SIZE))


def _topic_body(x_ref, eff_ref, rtd_ref, rtt_ref,
                wtd_ref, btd_ref, wtdc_ref, wtdn_ref,
                wtt_ref, btt_ref, wttc_ref, wttn_ref,
                ttdc_ref, ttdn_ref, tttc_ref, tttn_ref,
                otd_ref, ott_ref):
    x = x_ref[...]
    eff = eff_ref[...]
    zero = (eff == 0.0).astype(jnp.float32)
    mtd = (rtd_ref[...] < 0.1).astype(jnp.float32) * zero
    mtt = (rtt_ref[...] < 0.1).astype(jnp.float32) * zero

    def trans(m, t_ref):
        t = t_ref[...]
        return (m[:, 0:1] * t[0, 0] + m[:, 1:2] * t[0, 1]
                + m[:, 2:3] * t[0, 2])

    def mm(a, w_ref):
        return jnp.dot(a, w_ref[...].T, preferred_element_type=jnp.float32)

    ew_td = trans(eff, ttdc_ref)
    nw_td = trans(mtd, ttdn_ref)
    ew_tt = trans(eff, tttc_ref)
    nw_tt = trans(mtt, tttn_ref)
    otd_ref[...] = (mm(x, wtd_ref) + btd_ref[...]
                    + mm(x * ew_td, wtdc_ref) - mm(x * nw_td, wtdn_ref))
    ott_ref[...] = (mm(x, wtt_ref) + btt_ref[...]
                    + mm(x * ew_tt, wttc_ref) - mm(x * nw_tt, wttn_ref))


def _topic_proj(x, eff, rtd, rtt, wtd, btd, wtdc, wtdn,
                wtt, btt, wttc, wttn, ttdc, ttdn, tttc, tttn):
    blk = 1000
    mat = lambda: pl.BlockSpec((OUT_SIZE, OUT_SIZE), lambda i: (0, 0))
    tri = lambda: pl.BlockSpec((1, 3), lambda i: (0, 0))
    bia = lambda: pl.BlockSpec((1, OUT_SIZE), lambda i: (0, 0))
    return pl.pallas_call(
        _topic_body,
        grid=(N_TOPIC // blk,),
        in_specs=[
            pl.BlockSpec((blk, OUT_SIZE), lambda i: (i, 0)),
            pl.BlockSpec((blk, 3), lambda i: (i, 0)),
            pl.BlockSpec((blk, 3), lambda i: (i, 0)),
            pl.BlockSpec((blk, 3), lambda i: (i, 0)),
            mat(), bia(), mat(), mat(),
            mat(), bia(), mat(), mat(),
            tri(), tri(), tri(), tri(),
        ],
        out_specs=[pl.BlockSpec((blk, OUT_SIZE), lambda i: (i, 0))] * 2,
        out_shape=[jax.ShapeDtypeStruct((N_TOPIC, OUT_SIZE), jnp.float32)] * 2,
    )(x, eff, rtd, rtt, wtd, btd.reshape(1, -1), wtdc, wtdn,
      wtt, btt.reshape(1, -1), wttc, wttn, ttdc, ttdn, tttc, tttn)


_ONES32 = None


def _cnt_col(c_block):
    ones = jnp.ones((32, 1), jnp.float32)
    return lax.dot_general(c_block, ones, (((0,), (0,)), ((), ())),
                           preferred_element_type=jnp.float32)


def _word2_body(s_ref, c_ref, wwt_ref, bwt_ref, wwd_ref, bwd_ref,
                h_ref, owt_ref, owd_ref):
    c = jnp.maximum(_cnt_col(c_ref[...]), 1.0)
    h = s_ref[...] / c
    h_ref[...] = h
    owt_ref[...] = jnp.dot(h, wwt_ref[...].T,
                           preferred_element_type=jnp.float32) + bwt_ref[...]
    owd_ref[...] = jnp.dot(h, wwd_ref[...].T,
                           preferred_element_type=jnp.float32) + bwd_ref[...]


def _word_pass2(s, cnt, wwt, bwt, wwd, bwd):
    blk = 512
    mat = lambda: pl.BlockSpec((OUT_SIZE, OUT_SIZE), lambda i: (0, 0))
    bia = lambda: pl.BlockSpec((1, OUT_SIZE), lambda i: (0, 0))
    return pl.pallas_call(
        _word2_body,
        grid=(NP_WORD // blk,),
        in_specs=[
            pl.BlockSpec((blk, OUT_SIZE), lambda i: (i, 0)),
            pl.BlockSpec((32, blk), lambda i: (0, i)),
            mat(), bia(), mat(), bia(),
        ],
        out_specs=[pl.BlockSpec((blk, OUT_SIZE), lambda i: (i, 0))] * 3,
        out_shape=[jax.ShapeDtypeStruct((NP_WORD, OUT_SIZE), jnp.float32)] * 3,
    )(s, cnt, wwt, bwt.reshape(1, -1), wwd, bwd.reshape(1, -1))


def _combine2_body(sa0_ref, sa1_ref, ca_ref, sb0_ref, sb1_ref, cb_ref,
                   o_ref):
    ca = jnp.maximum(_cnt_col(ca_ref[...]), 1.0)
    cb = jnp.maximum(_cnt_col(cb_ref[...]), 1.0)
    o_ref[...] = ((sa0_ref[...] + sa1_ref[...]) / ca
                  + (sb0_ref[...] + sb1_ref[...]) / cb)


def _combine2(n_pad, sa0, sa1, ca, sb0, sb1, cb):
    blk = 512
    sb_spec = lambda: pl.BlockSpec((blk, OUT_SIZE), lambda i: (i, 0))
    cb_spec = lambda: pl.BlockSpec((32, blk), lambda i: (0, i))
    return pl.pallas_call(
        _combine2_body,
        grid=(n_pad // blk,),
        in_specs=[sb_spec(), sb_spec(), cb_spec(),
                  sb_spec(), sb_spec(), cb_spec()],
        out_specs=pl.BlockSpec((blk, OUT_SIZE), lambda i: (i, 0)),
        out_shape=jax.ShapeDtypeStruct((n_pad, OUT_SIZE), jnp.float32),
    )(sa0, sa1, ca, sb0, sb1, cb)


def _combine_body(sa_ref, ca_ref, sb_ref, cb_ref, o_ref):
    ca = jnp.maximum(_cnt_col(ca_ref[...]), 1.0)
    cb = jnp.maximum(_cnt_col(cb_ref[...]), 1.0)
    o_ref[...] = sa_ref[...] / ca + sb_ref[...] / cb


def _combine(n_pad, sa, ca, sb, cb):
    blk = 512
    return pl.pallas_call(
        _combine_body,
        grid=(n_pad // blk,),
        in_specs=[
            pl.BlockSpec((blk, OUT_SIZE), lambda i: (i, 0)),
            pl.BlockSpec((32, blk), lambda i: (0, i)),
            pl.BlockSpec((blk, OUT_SIZE), lambda i: (i, 0)),
            pl.BlockSpec((32, blk), lambda i: (0, i)),
        ],
        out_specs=pl.BlockSpec((blk, OUT_SIZE), lambda i: (i, 0)),
        out_shape=jax.ShapeDtypeStruct((n_pad, OUT_SIZE), jnp.float32),
    )(sa, ca, sb, cb)


def _mesh():
    return plsc.VectorSubcoreMesh(core_axis_name="c", subcore_axis_name="s")


def _edge_passes(tables, edges, out_shapes, passes, acc_rows, bb):
    nt = len(tables)
    ne = len(edges)
    no = len(out_shapes)

    @functools.partial(
        pl.kernel,
        out_type=[jax.ShapeDtypeStruct((n, D), jnp.float32)
                  for n in out_shapes],
        mesh=_mesh(),
        compiler_params=pltpu.CompilerParams(needs_layout_passes=False),
        scratch_types=[
            pltpu.VMEM_SHARED((acc_rows, D), jnp.float32),
            pltpu.VMEM((bb,), jnp.int32),
            pltpu.VMEM((bb,), jnp.int32),
            pltpu.VMEM((bb,), jnp.float32),
            pltpu.VMEM((bb, D), jnp.float32),
            pltpu.SemaphoreType.DMA,
            pltpu.SemaphoreType.DMA,
        ],
    )
    def seg_kernel(*refs):
        tabs = refs[:nt]
        srcs = refs[nt:nt + ne]
        dsts = refs[nt + ne:nt + 2 * ne]
        ws = refs[nt + 2 * ne:nt + 3 * ne]
        outs = refs[nt + 3 * ne:nt + 3 * ne + no]
        acc, src_v, dst_v, w_v, rows_v, sem, sem_i = refs[nt + 3 * ne + no:]
        cid = lax.axis_index("c")
        sid = lax.axis_index("s")

        for (core, ti, ei, oi, q0, rng, n_e, e0, e1) in passes:
            @pl.when(cid == core)
            def _(ti=ti, ei=ei, oi=oi, q0=q0, rng=rng, n_e=n_e,
                  e0=e0, e1=e1):
                tbl = tabs[ti]
                srch, dsth, wh = srcs[ei], dsts[ei], ws[ei]
                out = outs[oi]
                nb = (e1 - e0) // (NTILES * bb)
                zrows = rng + 16
                zpt = zrows // NTILES
                rpt = rng // NTILES

                def zr(i, _):
                    for c in range(D // 16):
                        rows_v[i, pl.ds(c * 16, 16)] = (
                            jnp.zeros((16,), jnp.float32))
                    return 0
                lax.fori_loop(0, bb, zr, 0)
                off = 0
                while off < zpt:
                    n = min(bb, zpt - off)
                    pltpu.sync_copy(rows_v.at[pl.ds(0, n)],
                                    acc.at[pl.ds(sid * zpt + off, n)])
                    off += n
                plsc.subcore_barrier()

                def batch(b, _):
                    base = e0 + (b * NTILES + sid) * bb

                    @pl.when(base < n_e)
                    def _():
                        d1 = pltpu.async_copy(
                            srch.at[pl.ds(base, bb)], src_v, sem_i)
                        d2 = pltpu.async_copy(
                            dsth.at[pl.ds(base, bb)], dst_v, sem_i)
                        d3 = pltpu.async_copy(
                            wh.at[pl.ds(base, bb)], w_v, sem_i)
                        d1.wait()
                        g2 = pltpu.async_copy(tbl.at[src_v], rows_v, sem)
                        d2.wait()

                        def remap(g, _):
                            dvec = dst_v[pl.ds(g * 16, 16)] - q0
                            ok = (dvec >= 0) & (dvec < rng)
                            dst_v[pl.ds(g * 16, 16)] = jnp.where(
                                ok, dvec, rng)
                            return 0
                        lax.fori_loop(0, bb // 16, remap, 0)
                        d3.wait()
                        g2.wait()

                        def scale(g, _):
                            wvec = w_v[pl.ds(g * 16, 16)]
                            for j in range(16):
                                wspl = jnp.broadcast_to(wvec[j], (16,))
                                e = g * 16 + j
                                for c in range(D // 16):
                                    rows_v[e, pl.ds(c * 16, 16)] = (
                                        rows_v[e, pl.ds(c * 16, 16)] * wspl)
                            return 0
                        lax.fori_loop(0, bb // 16, scale, 0)

                        pltpu.sync_copy(rows_v, acc.at[dst_v], add=True)
                    return 0
                lax.fori_loop(0, nb, batch, 0)
                plsc.subcore_barrier()
                pltpu.sync_copy(acc.at[pl.ds(sid * rpt, rpt)],
                                out.at[pl.ds(q0 + sid * rpt, rpt)])
                plsc.subcore_barrier()

    flat = list(tables)
    for i in range(3):
        flat += [e[i] for e in edges]
    res = seg_kernel(*flat)
    return res if isinstance(res, (tuple, list)) else (res,)


def _counts_kernel(dsts_padded, specs):
    ne = len(specs)
    npmax = max(n for (_, n) in specs)

    @functools.partial(
        pl.kernel,
        out_type=[jax.ShapeDtypeStruct((32, n), jnp.float32)
                  for (_, n) in specs],
        mesh=_mesh(),
        compiler_params=pltpu.CompilerParams(needs_layout_passes=False),
        scratch_types=[
            pltpu.VMEM((npmax,), jnp.float32),
            pltpu.VMEM((B,), jnp.int32),
        ],
    )
    def cnt_kernel(*refs):
        dsts = refs[:ne]
        outs = refs[ne:2 * ne]
        plane, dst_v = refs[2 * ne:]
        cid = lax.axis_index("c")
        sid = lax.axis_index("s")
        wid = sid * NCORES + cid
        lane = lax.iota(jnp.int32, 16)
        ones = jnp.ones((16,), jnp.float32)

        for k, (n_e, n_pad) in enumerate(specs):
            dsth = dsts[k]
            out = outs[k]
            ep = dsth.shape[0]
            nb = ep // (32 * B)

            def zp(i, _):
                plane[pl.ds(i * 16, 16)] = jnp.zeros((16,), jnp.float32)
                return 0
            lax.fori_loop(0, n_pad // 16, zp, 0)

            def batch(b, _):
                base = (b * 32 + wid) * B

                @pl.when(base < n_e)
                def _():
                    pltpu.sync_copy(dsth.at[pl.ds(base, B)], dst_v)

                    def grp(g, _):
                        dvec = dst_v[pl.ds(g * 16, 16)]
                        valid = (base + g * 16 + lane) < n_e
                        for l in range(16):
                            m = valid & (lane == l)
                            plsc.addupdate_scatter(
                                plane, [dvec], ones, mask=m)
                        return 0
                    lax.fori_loop(0, B // 16, grp, 0)
                return 0
            lax.fori_loop(0, nb, batch, 0)
            pltpu.sync_copy(plane.at[pl.ds(0, n_pad)], out.at[wid])

    res = cnt_kernel(*dsts_padded)
    return res if isinstance(res, (tuple, list)) else (res,)


WW_EP = 311296
WW_SL = WW_EP // 32
WW_CH = 128
WW_NCHUNK = WW_SL // WW_CH + 1
WW_CAP = WW_NCHUNK * WW_CH
WW_TOT = 4 * 32 * WW_CAP
QW = NP_WORD // 4


def _part_ww(src_p, dst_p, w_p):
    @functools.partial(
        pl.kernel,
        out_type=[jax.ShapeDtypeStruct((WW_TOT,), jnp.int32),
                  jax.ShapeDtypeStruct((WW_TOT,), jnp.int32),
                  jax.ShapeDtypeStruct((WW_TOT,), jnp.float32),
                  jax.ShapeDtypeStruct((512,), jnp.int32)],
        mesh=_mesh(),
        compiler_params=pltpu.CompilerParams(needs_layout_passes=False),
        scratch_types=[
            pltpu.VMEM((512,), jnp.int32),
            pltpu.VMEM((512,), jnp.int32),
            pltpu.VMEM((512,), jnp.float32),
            pltpu.VMEM((4 * 2 * (WW_CH + 16),), jnp.int32),
            pltpu.VMEM((4 * 2 * (WW_CH + 16),), jnp.int32),
            pltpu.VMEM((4 * 2 * (WW_CH + 16),), jnp.float32),
            pltpu.VMEM((16,), jnp.int32),
            pltpu.SemaphoreType.DMA,
        ],
    )
    def part(srch, dsth, wh, osrc, odst, ow, ocnt,
             sv, dv, wv, qs, qd, qwb, cv, semp):
        cid = lax.axis_index("c")
        sid = lax.axis_index("s")
        wid = sid * NCORES + cid
        lane = lax.iota(jnp.int32, 16)
        base0 = wid * WW_SL

        def zq(i, _):
            qs[pl.ds(i * 16, 16)] = jnp.zeros((16,), jnp.int32)
            qd[pl.ds(i * 16, 16)] = jnp.zeros((16,), jnp.int32)
            qwb[pl.ds(i * 16, 16)] = jnp.zeros((16,), jnp.float32)
            return 0
        lax.fori_loop(0, 4 * 2 * (WW_CH + 16) // 16, zq, 0)
        HS = WW_CH + 16
        QB = 2 * HS

        def batch(b, carry):
            d1 = pltpu.async_copy(
                srch.at[pl.ds(base0 + b * 512, 512)], sv, semp)
            d2 = pltpu.async_copy(
                dsth.at[pl.ds(base0 + b * 512, 512)], dv, semp)
            d3 = pltpu.async_copy(
                wh.at[pl.ds(base0 + b * 512, 512)], wv, semp)
            d1.wait()
            d2.wait()
            d3.wait()

            def grp(g, c2):
                svec = sv[pl.ds(g * 16, 16)]
                dvec = dv[pl.ds(g * 16, 16)]
                wvec = wv[pl.ds(g * 16, 16)]
                gidx = base0 + b * 512 + g * 16 + lane
                valid = gidx < E_WW
                new = []
                for q in range(4):
                    pos, nf = c2[2 * q], c2[2 * q + 1]
                    m = valid & (dvec >= q * QW) & (dvec < (q + 1) * QW)
                    hb = q * QB + (nf & 1) * HS
                    plsc.store_compressed(
                        qs.at[pl.ds(hb + pos, 16)], svec, mask=m)
                    plsc.store_compressed(
                        qd.at[pl.ds(hb + pos, 16)], dvec, mask=m)
                    plsc.store_compressed(
                        qwb.at[pl.ds(hb + pos, 16)], wvec, mask=m)
                    pos = pos + plsc.all_reduce_population_count(m)[0]
                    full = pos >= WW_CH

                    @pl.when(full)
                    def _(q=q, nf=nf, hb=hb):
                        ob = q * QB + (HS - (nf & 1) * HS)
                        qs[pl.ds(ob, 16)] = qs[pl.ds(hb + WW_CH, 16)]
                        qd[pl.ds(ob, 16)] = qd[pl.ds(hb + WW_CH, 16)]
                        qwb[pl.ds(ob, 16)] = qwb[pl.ds(hb + WW_CH, 16)]
                        dst0 = (q * 32 + wid) * WW_CAP + nf * WW_CH
                        pltpu.sync_copy(qs.at[pl.ds(hb, WW_CH)],
                                        osrc.at[pl.ds(dst0, WW_CH)])
                        pltpu.sync_copy(qd.at[pl.ds(hb, WW_CH)],
                                        odst.at[pl.ds(dst0, WW_CH)])
                        pltpu.sync_copy(qwb.at[pl.ds(hb, WW_CH)],
                                        ow.at[pl.ds(dst0, WW_CH)])

                    pos = jnp.where(full, pos - WW_CH, pos)
                    nf = nf + full.astype(jnp.int32)
                    new += [pos, nf]
                return tuple(new)
            return lax.fori_loop(0, 32, grp, carry)

        z = jnp.int32(0)
        carry = lax.fori_loop(0, WW_SL // 512, batch,
                              (z, z, z, z, z, z, z, z))

        cvec = jnp.zeros((16,), jnp.int32)
        for q in range(4):
            pos, nf = carry[2 * q], carry[2 * q + 1]

            @pl.when(pos > 0)
            def _(q=q, nf=nf):
                hb = q * QB + (nf & 1) * HS
                dst0 = (q * 32 + wid) * WW_CAP + nf * WW_CH
                pltpu.sync_copy(qs.at[pl.ds(hb, WW_CH)],
                                osrc.at[pl.ds(dst0, WW_CH)])
                pltpu.sync_copy(qd.at[pl.ds(hb, WW_CH)],
                                odst.at[pl.ds(dst0, WW_CH)])
                pltpu.sync_copy(qwb.at[pl.ds(hb, WW_CH)],
                                ow.at[pl.ds(dst0, WW_CH)])

            total = nf * WW_CH + pos
            cvec = jnp.where(lane == q, jnp.broadcast_to(total, (16,)), cvec)
        cv[...] = cvec
        pltpu.sync_copy(cv, ocnt.at[pl.ds(wid * 16, 16)])

    return part(src_p, dst_p, w_p)


def _seg_ww(tbl, psrc, pdst, pw, cnts):
    @functools.partial(
        pl.kernel,
        out_type=jax.ShapeDtypeStruct((NP_WORD, D), jnp.float32),
        mesh=_mesh(),
        compiler_params=pltpu.CompilerParams(needs_layout_passes=False),
        scratch_types=[
            pltpu.VMEM_SHARED((QW + 16, D), jnp.float32),
            pltpu.VMEM((WW_CH,), jnp.int32),
            pltpu.VMEM((WW_CH,), jnp.int32),
            pltpu.VMEM((WW_CH,), jnp.float32),
            pltpu.VMEM((WW_CH, D), jnp.float32),
            pltpu.VMEM((512,), jnp.int32),
            pltpu.SemaphoreType.DMA,
            pltpu.SemaphoreType.DMA,
        ],
    )
    def seg(tblh, psh, pdh, pwh, ch, out,
            acc, src_v, dst_v, w_v, rows_v, cnt_v, sem, sem_i):
        cid = lax.axis_index("c")
        sid = lax.axis_index("s")
        lane = lax.iota(jnp.int32, 16)
        pltpu.sync_copy(ch, cnt_v)
        for q in range(4):
            @pl.when(cid == q // 2)
            def _(q=q):
                q0 = q * QW
                zpt = (QW + 16) // NTILES
                rpt = QW // NTILES

                def zr(i, _):
                    for c in range(D // 16):
                        rows_v[i, pl.ds(c * 16, 16)] = (
                            jnp.zeros((16,), jnp.float32))
                    return 0
                lax.fori_loop(0, WW_CH, zr, 0)
                off = 0
                while off < zpt:
                    n = min(WW_CH, zpt - off)
                    pltpu.sync_copy(rows_v.at[pl.ds(0, n)],
                                    acc.at[pl.ds(sid * zpt + off, n)])
                    off += n
                plsc.subcore_barrier()

                for k in range(2):
                    st = sid * 2 + k
                    cnt = cnt_v[pl.ds(st * 16, 16)][q]
                    nch = (cnt + WW_CH - 1) // WW_CH
                    lbase = (q * 32 + st) * WW_CAP

                    def chunk(c, _):
                        cb = lbase + c * WW_CH
                        d1 = pltpu.async_copy(
                            psh.at[pl.ds(cb, WW_CH)], src_v, sem_i)
                        d2 = pltpu.async_copy(
                            pdh.at[pl.ds(cb, WW_CH)], dst_v, sem_i)
                        d3 = pltpu.async_copy(
                            pwh.at[pl.ds(cb, WW_CH)], w_v, sem_i)
                        rem = cnt - c * WW_CH
                        d1.wait()
                        g = pltpu.async_copy(tblh.at[src_v], rows_v, sem)
                        d2.wait()

                        def remap(gi, _):
                            dvec = dst_v[pl.ds(gi * 16, 16)] - q0
                            ok = (((gi * 16 + lane) < rem)
                                  & (dvec >= 0) & (dvec < QW))
                            dst_v[pl.ds(gi * 16, 16)] = jnp.where(
                                ok, dvec, QW)
                            return 0
                        lax.fori_loop(0, WW_CH // 16, remap, 0)
                        d3.wait()
                        g.wait()

                        def scale(g, _):
                            wvec = w_v[pl.ds(g * 16, 16)]
                            for j in range(16):
                                wspl = jnp.broadcast_to(wvec[j], (16,))
                                e = g * 16 + j
                                for c2 in range(D // 16):
                                    rows_v[e, pl.ds(c2 * 16, 16)] = (
                                        rows_v[e, pl.ds(c2 * 16, 16)] * wspl)
                            return 0
                        lax.fori_loop(0, WW_CH // 16, scale, 0)

                        pltpu.sync_copy(rows_v, acc.at[dst_v], add=True)
                        return 0
                    lax.fori_loop(0, nch, chunk, 0)
                plsc.subcore_barrier()
                pltpu.sync_copy(acc.at[pl.ds(sid * rpt, rpt)],
                                out.at[pl.ds(q0 + sid * rpt, rpt)])
                plsc.subcore_barrier()

    return seg(tbl, psrc, pdst, pw, cnts)


def _pad_edges(src, dst, w):
    e = src.shape[0]
    ep = _rup(e, 32 * B)
    pad = ep - e
    src_p = jnp.pad(src.astype(jnp.int32), (0, pad))
    dst_p = jnp.pad(dst.astype(jnp.int32), (0, pad))
    w_p = jnp.pad(w, (0, pad))
    return src_p, dst_p, w_p


def kernel(feat_word, feat_topic, effect,
           ww_src, ww_dst, ww_w, wt_src, wt_dst, wt_w,
           wd_src, wd_dst, wd_w, td_src, td_dst, td_w,
           tt_src, tt_dst, tt_w, rand_td, rand_tt,
           W_ww, b_ww, W_wt, b_wt, W_wd, b_wd, W_td, b_td, W_tt, b_tt,
           W_td_cau, W_td_noi, W_tt_cau, W_tt_noi,
           W_td_cau_trans, W_td_noi_trans, W_tt_cau_trans, W_tt_noi_trans):
    ww = _pad_edges(ww_src, ww_dst, ww_w)
    wt = _pad_edges(wt_src, wt_dst, wt_w)
    wd = _pad_edges(wd_src, wd_dst, wd_w)
    td = _pad_edges(td_src, td_dst, td_w)
    tt = _pad_edges(tt_src, tt_dst, tt_w)

    c_ww, c_wt, c_tt, c_wd, c_td = _counts_kernel(
        [ww[1], wt[1], tt[1], wd[1], td[1]],
        [(E_WW, NP_WORD), (E_WT, NP_TOPIC), (E_TT, NP_TOPIC),
         (E_WD, NP_DOC), (E_TD, NP_DOC)])

    tww = _proj_word(feat_word, W_ww, b_ww)
    pws, pwd, pww, wcnt = _part_ww(*ww)
    s_ww = _seg_ww(tww, pws, pwd, pww, wcnt)

    ttd, ttt = _topic_proj(feat_topic, effect, rand_td, rand_tt,
                           W_td, b_td, W_td_cau, W_td_noi,
                           W_tt, b_tt, W_tt_cau, W_tt_noi,
                           W_td_cau_trans, W_td_noi_trans,
                           W_tt_cau_trans, W_tt_noi_trans)

    h_word, twt, twd = _word_pass2(s_ww, c_ww, W_wt, b_wt, W_wd, b_wd)

    wt_h = wt[0].shape[0] // 2
    tt_h = tt[0].shape[0] // 2
    s_wt0, s_wt1, s_tt0, s_tt1 = _edge_passes(
        [twt, ttt], [wt, tt], [NP_TOPIC] * 4,
        [(0, 0, 0, 0, 0, NP_TOPIC, E_WT, 0, wt_h),
         (1, 0, 0, 1, 0, NP_TOPIC, E_WT, wt_h, 2 * wt_h),
         (0, 1, 1, 2, 0, NP_TOPIC, E_TT, 0, tt_h),
         (1, 1, 1, 3, 0, NP_TOPIC, E_TT, tt_h, 2 * tt_h)],
        NP_TOPIC + 16, 512)

    HD = NP_DOC // 2
    wd_e = wd[0].shape[0]
    td_e = td[0].shape[0]
    s_wd, s_td = _edge_passes(
        [twd, ttd], [wd, td], [NP_DOC, NP_DOC],
        [(0, 0, 0, 0, 0, HD, E_WD, 0, wd_e), (1, 0, 0, 0, HD, HD, E_WD, 0, wd_e),
         (0, 1, 1, 1, 0, HD, E_TD, 0, td_e), (1, 1, 1, 1, HD, HD, E_TD, 0, td_e)],
        HD + 16, 256)

    h_topic = _combine2(NP_TOPIC, s_wt0, s_wt1, c_wt, s_tt0, s_tt1, c_tt)
    h_doc = _combine(NP_DOC, s_wd, c_wd, s_td, c_td)

    return (h_word[:N_WORD], h_topic[:N_TOPIC], h_doc[:N_DOC])
